# Initial kernel scaffold; baseline (speedup 1.0000x reference)
import jax
import jax.numpy as jnp
from jax.experimental import pallas as pl


def kernel(x, params, edge_index):
    raise NotImplementedError("write your pallas kernel here")


# speedup vs baseline: 1.0000x; 1.0000x over previous
import jax
import jax.numpy as jnp
from jax.experimental import pallas as pl
from jax.experimental.pallas import tpu as pltpu

HEADS = 8
OC = 64


def _cfgs_for(num_layers, in_ch, hid, out, heads):
    cfgs = []
    cur = in_ch
    for i in range(num_layers):
        last = (i == num_layers - 1)
        oc = out if last else hid
        concat = (not last)
        da = oc * heads if concat else oc
        cfgs.append(dict(in_dim=cur, out_ch=oc, heads=heads, concat=concat,
                         dim_after=da, has_proj=(cur != da)))
        cur = da
    return cfgs


def _gat_conv(x, src, dst, p, c):
    N = x.shape[0]
    h = (x @ p["W"]).reshape(N, c["heads"], c["out_ch"])
    a_src = jnp.sum(h * p["att_src"], axis=-1)
    a_dst = jnp.sum(h * p["att_dst"], axis=-1)
    e = jax.nn.leaky_relu(a_src[src] + a_dst[dst], negative_slope=0.2)
    m = jax.ops.segment_max(e, dst, num_segments=N)
    ex = jnp.exp(e - m[dst])
    s = jax.ops.segment_sum(ex, dst, num_segments=N)
    alpha = ex / (s[dst] + 1e-16)
    out = jax.ops.segment_sum(h[src] * alpha[:, :, None], dst, num_segments=N)
    if c["concat"]:
        out = out.reshape(N, c["heads"] * c["out_ch"])
    else:
        out = out.mean(axis=1)
    return out + p["bias"]


def _layer_norm(x, g, b):
    mu = jnp.mean(x, axis=-1, keepdims=True)
    var = jnp.var(x, axis=-1, keepdims=True)
    return (x - mu) / jnp.sqrt(var + 1e-5) * g + b


def _final_mm_kernel(h_ref, w_ref, b_ref, o_ref):
    o_ref[...] = h_ref[...] @ w_ref[...] + b_ref[...]


def _final_mm(h, W2, b2):
    n, k = h.shape
    m = W2.shape[0]
    return pl.pallas_call(
        _final_mm_kernel,
        out_shape=jax.ShapeDtypeStruct((n, m), jnp.float32),
    )(h, W2.T, b2.reshape(1, m))


def kernel(x, params, edge_index):
    N = x.shape[0]
    loop = jnp.arange(N, dtype=edge_index.dtype)
    src = jnp.concatenate([edge_index[0], loop])
    dst = jnp.concatenate([edge_index[1], loop])
    feats = []
    for layers, nl in zip(params["encoders"], (2, 3)):
        cfgs = _cfgs_for(nl, 128, 64, 64, HEADS)
        h = x
        for i, (p, c) in enumerate(zip(layers, cfgs)):
            h_in = h
            xg = _gat_conv(h_in, src, dst, p, c)
            xn = _layer_norm(xg, p["ln_g"], p["ln_b"])
            proj = h_in @ p["proj_W"].T + p["proj_b"] if c["has_proj"] else h_in
            xr = xn + jax.nn.sigmoid(p["res_w"]) * proj
            h = jax.nn.elu(xr) if i < len(cfgs) - 1 else xr
        feats.append(h)
    x0 = feats[0]
    pool = params["pool"]
    scores = jnp.tanh(x0 @ pool["W1"].T + pool["b1"]) @ pool["W2"].T + pool["b2"]
    w = jax.nn.softmax(scores, axis=0)
    g = jnp.sum(x0 * w, axis=0, keepdims=True)
    ge = jnp.broadcast_to(g, (x.shape[0], g.shape[1]))
    cat = jnp.concatenate(feats + [ge], axis=-1)
    fu = params["fusion"]
    h = jax.nn.relu(cat @ fu["W1"].T + fu["b1"])
    return _final_mm(h, fu["W2"], fu["b2"])


# speedup vs baseline: 21.7014x; 21.7007x over previous
import functools

import jax
import jax.numpy as jnp
from jax import lax
from jax.experimental import pallas as pl
from jax.experimental.pallas import tpu as pltpu
from jax.experimental.pallas import tpu_sc as plsc

N = 10000
NPAD = 10240
HEADS = 8
OC = 64
HD = HEADS * OC
NG = 4
C = 128
NTILES = 32
ROWS_PER_TILE = NPAD // 16
F32 = jnp.float32


def _cfgs_for(num_layers, in_ch=128, hid=64, out=64, heads=HEADS):
    cfgs = []
    cur = in_ch
    for i in range(num_layers):
        last = (i == num_layers - 1)
        oc = out if last else hid
        concat = (not last)
        da = oc * heads if concat else oc
        cfgs.append(dict(in_dim=cur, out_ch=oc, heads=heads, concat=concat,
                         dim_after=da, has_proj=(cur != da)))
        cur = da
    return cfgs


def _pre_call(xp, W, att_s, att_d, projWt, projb):
    ind = xp.shape[1]
    has_proj = projWt is not None
    BN = 1280
    grid = (NPAD // BN,)

    def body(x_ref, w_ref, as_ref, ad_ref, *rest):
        if has_proj:
            pw_ref, pb_ref = rest[0], rest[1]
            outs = rest[2:]
        else:
            outs = rest
        h0, h1, h2, h3, abs_ref, abd_ref = outs[:6]
        xb = x_ref[...]
        hb = jnp.dot(xb, w_ref[...], preferred_element_type=F32)
        acols, bcols = [], []
        for k in range(HEADS):
            hk = hb[:, OC * k:OC * (k + 1)]
            acols.append(jnp.sum(hk * as_ref[:, OC * k:OC * (k + 1)], axis=1,
                                 keepdims=True))
            bcols.append(jnp.sum(hk * ad_ref[:, OC * k:OC * (k + 1)], axis=1,
                                 keepdims=True))
        zeros8 = jnp.zeros((BN, 8), F32)
        abs_ref[...] = jnp.concatenate(acols + [zeros8], axis=1)
        abd_ref[...] = jnp.concatenate(bcols + [zeros8], axis=1)
        for g, hg in enumerate((h0, h1, h2, h3)):
            hg[...] = hb[:, 128 * g:128 * (g + 1)]
        if has_proj:
            da = pw_ref.shape[1]
            outs[6][...] = jnp.dot(xb, pw_ref[...],
                                   preferred_element_type=F32) + pb_ref[...]

    in_specs = [
        pl.BlockSpec((BN, ind), lambda i: (i, 0)),
        pl.BlockSpec((ind, HD), lambda i: (0, 0)),
        pl.BlockSpec((1, HD), lambda i: (0, 0)),
        pl.BlockSpec((1, HD), lambda i: (0, 0)),
    ]
    args = [xp, W, att_s, att_d]
    out_shape = [jax.ShapeDtypeStruct((NPAD, 128), F32) for _ in range(4)]
    out_shape += [jax.ShapeDtypeStruct((NPAD, 16), F32) for _ in range(2)]
    out_specs = [pl.BlockSpec((BN, 128), lambda i: (i, 0)) for _ in range(4)]
    out_specs += [pl.BlockSpec((BN, 16), lambda i: (i, 0)) for _ in range(2)]
    if has_proj:
        da = projWt.shape[1]
        in_specs += [pl.BlockSpec((ind, da), lambda i: (0, 0)),
                     pl.BlockSpec((1, da), lambda i: (0, 0))]
        args += [projWt, projb]
        out_shape.append(jax.ShapeDtypeStruct((NPAD, da), F32))
        out_specs.append(pl.BlockSpec((BN, da), lambda i: (i, 0)))
    return pl.pallas_call(
        body, grid=grid, in_specs=in_specs, out_specs=out_specs,
        out_shape=out_shape)(*args)


def _dyn_gather(row, idx):
    return lax.gather(
        row, idx[:, None],
        lax.GatherDimensionNumbers(offset_dims=(), collapsed_slice_dims=(0,),
                                   start_index_map=(0,)),
        slice_sizes=(1,), mode=lax.GatherScatterMode.PROMISE_IN_BOUNDS)


def _edge_call(hgs, ab_s, ab_d, srcp, dstp, epad, q):
    mesh = plsc.VectorSubcoreMesh(core_axis_name="c", subcore_axis_name="s")
    out_type = [
        jax.ShapeDtypeStruct((2, NG, NPAD, 128), F32),
        jax.ShapeDtypeStruct((2, NPAD, 16), F32),
        jax.ShapeDtypeStruct((epad, 16), F32),
    ]
    scratch = [
        pltpu.VMEM((C,), jnp.int32),
        pltpu.VMEM((C,), jnp.int32),
        pltpu.VMEM((C, 16), F32),
        pltpu.VMEM((C, 16), F32),
        pltpu.VMEM((C, 16), F32),
        pltpu.VMEM((C, 128), F32),
        pltpu.SemaphoreType.DMA,
        pltpu.VMEM_SHARED((NPAD, 128), F32),
        pltpu.VMEM_SHARED((NPAD, 16), F32),
    ]

    @functools.partial(pl.kernel, mesh=mesh, out_type=out_type,
                       scratch_types=scratch,
                       compiler_params=pltpu.CompilerParams(
                           use_tc_tiling_on_sc=False))
    def k(h0, h1, h2, h3, absr, abdr, srcr, dstr,
          accs, ssum, wg,
          idxs, idxd, Sv, Dv, Wv, rows, sem, acc_sp, s_sp):
        cid = lax.axis_index("c")
        sid = lax.axis_index("s")
        t = cid * 16 + sid
        r0 = sid * ROWS_PER_TILE

        def zw(e, _):
            Wv[e, :] = jnp.zeros((16,), F32)
            return 0
        lax.fori_loop(0, C, zw, 0)
        for part in range(ROWS_PER_TILE // C):
            pltpu.sync_copy(Wv, s_sp.at[pl.ds(r0 + C * part, C)])
        plsc.subcore_barrier()

        def chunk1(i, _):
            base = (t * q + i) * C
            pltpu.sync_copy(srcr.at[pl.ds(base, C)], idxs)
            pltpu.sync_copy(dstr.at[pl.ds(base, C)], idxd)
            pltpu.async_copy(absr.at[idxs], Sv, sem).wait()
            pltpu.async_copy(abdr.at[idxd], Dv, sem).wait()

            def edge(e, _):
                z = Sv[e, :] + Dv[e, :]
                Wv[e, :] = jnp.exp(jnp.maximum(z, 0.2 * z))
                return 0
            lax.fori_loop(0, C, edge, 0)
            pltpu.sync_copy(Wv, s_sp.at[idxd], add=True)
            pltpu.sync_copy(Wv, wg.at[pl.ds(base, C)])
            return 0
        lax.fori_loop(0, q, chunk1, 0)
        plsc.subcore_barrier()
        pltpu.sync_copy(s_sp.at[pl.ds(r0, ROWS_PER_TILE)],
                        ssum.at[cid, pl.ds(r0, ROWS_PER_TILE)])

        for g in range(NG):
            hg = (h0, h1, h2, h3)[g]

            def zr(e, _):
                for j in range(8):
                    rows[e, pl.ds(16 * j, 16)] = jnp.zeros((16,), F32)
                return 0
            lax.fori_loop(0, C, zr, 0)
            for part in range(ROWS_PER_TILE // C):
                pltpu.sync_copy(rows, acc_sp.at[pl.ds(r0 + C * part, C)])
            plsc.subcore_barrier()

            gi0 = jnp.full((16,), 2 * g, jnp.int32)
            gi1 = jnp.full((16,), 2 * g + 1, jnp.int32)

            def chunk2(i, _):
                base = (t * q + i) * C
                pltpu.sync_copy(srcr.at[pl.ds(base, C)], idxs)
                pltpu.sync_copy(dstr.at[pl.ds(base, C)], idxd)
                pltpu.sync_copy(wg.at[pl.ds(base, C)], Wv)
                pltpu.async_copy(hg.at[idxs], rows, sem).wait()

                def edge(e, _):
                    wrow = Wv[e, :]
                    b0 = _dyn_gather(wrow, gi0)
                    b1 = _dyn_gather(wrow, gi1)
                    for j in range(4):
                        rows[e, pl.ds(16 * j, 16)] = \
                            rows[e, pl.ds(16 * j, 16)] * b0
                    for j in range(4, 8):
                        rows[e, pl.ds(16 * j, 16)] = \
                            rows[e, pl.ds(16 * j, 16)] * b1
                    return 0
                lax.fori_loop(0, C, edge, 0)
                pltpu.sync_copy(rows, acc_sp.at[idxd], add=True)
                return 0
            lax.fori_loop(0, q, chunk2, 0)
            plsc.subcore_barrier()
            pltpu.sync_copy(acc_sp.at[pl.ds(r0, ROWS_PER_TILE)],
                            accs.at[cid, g, pl.ds(r0, ROWS_PER_TILE)])

    accs, ssum, _ = k(hgs[0], hgs[1], hgs[2], hgs[3], ab_s, ab_d, srcp, dstp)
    return accs, ssum


def _post_call(accs, ssum, proj, bias, ln_g, ln_b, sw, concat, last):
    da = HD if concat else OC
    BN = 1280
    grid = (NPAD // BN,)

    def body(accs_ref, ssum_ref, proj_ref, bias_ref, g_ref, b_ref, sw_ref,
             out_ref):
        s = ssum_ref[0] + ssum_ref[1]
        cols = []
        for k in range(HEADS):
            g, m = k // 2, k % 2
            a = (accs_ref[0, g][:, OC * m:OC * (m + 1)]
                 + accs_ref[1, g][:, OC * m:OC * (m + 1)])
            den = s[:, k:k + 1] + 1e-16
            cols.append(a / den)
        if concat:
            xg = jnp.concatenate(cols, axis=1)
        else:
            acc = cols[0]
            for ck in cols[1:]:
                acc = acc + ck
            xg = acc / float(HEADS)
        xg = xg + bias_ref[...]
        mu = jnp.mean(xg, axis=1, keepdims=True)
        var = jnp.mean((xg - mu) * (xg - mu), axis=1, keepdims=True)
        xn = (xg - mu) * lax.rsqrt(var + 1e-5) * g_ref[...] + b_ref[...]
        res = xn + sw_ref[...] * proj_ref[...]
        if last:
            out_ref[...] = res
        else:
            out_ref[...] = jnp.where(res > 0, res, jnp.exp(res) - 1.0)

    in_specs = [
        pl.BlockSpec((2, NG, BN, 128), lambda i: (0, 0, i, 0)),
        pl.BlockSpec((2, BN, 16), lambda i: (0, i, 0)),
        pl.BlockSpec((BN, da), lambda i: (i, 0)),
        pl.BlockSpec((1, da), lambda i: (0, 0)),
        pl.BlockSpec((1, da), lambda i: (0, 0)),
        pl.BlockSpec((1, da), lambda i: (0, 0)),
        pl.BlockSpec((1, 1), lambda i: (0, 0)),
    ]
    return pl.pallas_call(
        body, grid=grid, in_specs=in_specs,
        out_specs=pl.BlockSpec((BN, da), lambda i: (i, 0)),
        out_shape=jax.ShapeDtypeStruct((NPAD, da), F32),
    )(accs, ssum, proj, bias, ln_g, ln_b, sw)


def _fusion_call(f0, f1, pw1t, pb1, pw2t, pb2, wa, wb, wgc, fb1, w2t, fb2):
    def body(f0r, f1r, pw1r, pb1r, pw2r, pb2r, war, wbr, wgr, fb1r, w2r,
             fb2r, out_ref):
        x0 = f0r[...]
        t1 = jnp.tanh(jnp.dot(x0, pw1r[...], preferred_element_type=F32)
                      + pb1r[...])
        sc = jnp.dot(t1, pw2r[...], preferred_element_type=F32) + pb2r[...]
        rid = lax.broadcasted_iota(jnp.int32, (NPAD, 1), 0)
        valid = rid < N
        scm = jnp.where(valid, sc, jnp.full_like(sc, -1e30))
        m = jnp.max(scm)
        ex = jnp.where(valid, jnp.exp(sc - m), jnp.zeros_like(sc))
        wgt = ex / jnp.sum(ex)
        gvec = jnp.sum(x0 * wgt, axis=0, keepdims=True)
        h1 = (jnp.dot(x0, war[...], preferred_element_type=F32)
              + jnp.dot(f1r[...], wbr[...], preferred_element_type=F32)
              + jnp.dot(gvec, wgr[...], preferred_element_type=F32)
              + fb1r[...])
        h1 = jnp.maximum(h1, 0.0)
        out_ref[...] = jnp.dot(h1, w2r[...],
                               preferred_element_type=F32) + fb2r[...]

    return pl.pallas_call(
        body,
        out_shape=jax.ShapeDtypeStruct((NPAD, 128), F32),
    )(f0, f1, pw1t, pb1, pw2t, pb2, wa, wb, wgc, fb1, w2t, fb2)


def _layer(h_in, p, c, srcp, dstp, epad, q, last):
    att_s = p["att_src"].reshape(1, HD)
    att_d = p["att_dst"].reshape(1, HD)
    if c["has_proj"]:
        projWt = p["proj_W"].T
        projb = p["proj_b"].reshape(1, -1)
        pre = _pre_call(h_in, p["W"], att_s, att_d, projWt, projb)
        hgs, ab_s, ab_d, proj = pre[:4], pre[4], pre[5], pre[6]
    else:
        pre = _pre_call(h_in, p["W"], att_s, att_d, None, None)
        hgs, ab_s, ab_d, proj = pre[:4], pre[4], pre[5], h_in
    accs, ssum = _edge_call(hgs, ab_s, ab_d, srcp, dstp, epad, q)
    sw = jax.nn.sigmoid(p["res_w"]).reshape(1, 1)
    return _post_call(accs, ssum, proj, p["bias"].reshape(1, -1),
                      p["ln_g"].reshape(1, -1), p["ln_b"].reshape(1, -1),
                      sw, c["concat"], last)


def kernel(x, params, edge_index):
    xp = jnp.pad(x, ((0, NPAD - N), (0, 0)))
    loop = jnp.arange(N, dtype=edge_index.dtype)
    src = jnp.concatenate([edge_index[0], loop]).astype(jnp.int32)
    dst = jnp.concatenate([edge_index[1], loop]).astype(jnp.int32)
    ne = src.shape[0]
    q = -(-ne // (NTILES * C))
    epad = NTILES * q * C
    npe = epad - ne
    pad_idx = (N + (jnp.arange(npe, dtype=jnp.int32) % 16)).astype(jnp.int32)
    srcp = jnp.concatenate([src, pad_idx])
    dstp = jnp.concatenate([dst, pad_idx])

    encs = params["encoders"]
    cfgs = [_cfgs_for(2), _cfgs_for(3)]
    state = [xp, xp]
    for li in range(3):
        for enc in range(2):
            if li >= len(cfgs[enc]):
                continue
            c = cfgs[enc][li]
            last = (li == len(cfgs[enc]) - 1)
            state[enc] = _layer(state[enc], encs[enc][li], c, srcp, dstp,
                                epad, q, last)

    pool = params["pool"]
    fu = params["fusion"]
    W1 = fu["W1"]
    out = _fusion_call(
        state[0], state[1],
        pool["W1"].T, pool["b1"].reshape(1, -1),
        pool["W2"].T, pool["b2"].reshape(1, -1),
        W1[:, :OC].T, W1[:, OC:2 * OC].T, W1[:, 2 * OC:].T,
        fu["b1"].reshape(1, -1), fu["W2"].T, fu["b2"].reshape(1, -1))
    return out[:N]


# speedup vs baseline: 34.0945x; 1.5711x over previous
import functools

import jax
import jax.numpy as jnp
from jax import lax
from jax.experimental import pallas as pl
from jax.experimental.pallas import tpu as pltpu
from jax.experimental.pallas import tpu_sc as plsc

N = 10000
NPAD = 10240
HEADS = 8
OC = 64
HD = HEADS * OC
NG = 4
C = 128
NTILES = 32
ROWS_PER_TILE = NPAD // 16
F32 = jnp.float32


def _cfgs_for(num_layers, in_ch=128, hid=64, out=64, heads=HEADS):
    cfgs = []
    cur = in_ch
    for i in range(num_layers):
        last = (i == num_layers - 1)
        oc = out if last else hid
        concat = (not last)
        da = oc * heads if concat else oc
        cfgs.append(dict(in_dim=cur, out_ch=oc, heads=heads, concat=concat,
                         dim_after=da, has_proj=(cur != da)))
        cur = da
    return cfgs


def _pre_call(xp, W, att_s, att_d, projWt, projb):
    ind = xp.shape[1]
    has_proj = projWt is not None
    BN = 1280
    grid = (NPAD // BN,)

    def body(x_ref, w_ref, as_ref, ad_ref, *rest):
        if has_proj:
            pw_ref, pb_ref = rest[0], rest[1]
            outs = rest[2:]
        else:
            outs = rest
        h0, h1, h2, h3, abs_ref, abd_ref = outs[:6]
        xb = x_ref[...]
        hb = jnp.dot(xb, w_ref[...], preferred_element_type=F32)
        acols, bcols = [], []
        for k in range(HEADS):
            hk = hb[:, OC * k:OC * (k + 1)]
            acols.append(jnp.sum(hk * as_ref[:, OC * k:OC * (k + 1)], axis=1,
                                 keepdims=True))
            bcols.append(jnp.sum(hk * ad_ref[:, OC * k:OC * (k + 1)], axis=1,
                                 keepdims=True))
        zeros8 = jnp.zeros((BN, 8), F32)
        abs_ref[...] = jnp.concatenate(acols + [zeros8], axis=1)
        abd_ref[...] = jnp.concatenate(bcols + [zeros8], axis=1)
        for g, hg in enumerate((h0, h1, h2, h3)):
            hg[...] = hb[:, 128 * g:128 * (g + 1)]
        if has_proj:
            da = pw_ref.shape[1]
            outs[6][...] = jnp.dot(xb, pw_ref[...],
                                   preferred_element_type=F32) + pb_ref[...]

    in_specs = [
        pl.BlockSpec((BN, ind), lambda i: (i, 0)),
        pl.BlockSpec((ind, HD), lambda i: (0, 0)),
        pl.BlockSpec((1, HD), lambda i: (0, 0)),
        pl.BlockSpec((1, HD), lambda i: (0, 0)),
    ]
    args = [xp, W, att_s, att_d]
    out_shape = [jax.ShapeDtypeStruct((NPAD, 128), F32) for _ in range(4)]
    out_shape += [jax.ShapeDtypeStruct((NPAD, 16), F32) for _ in range(2)]
    out_specs = [pl.BlockSpec((BN, 128), lambda i: (i, 0)) for _ in range(4)]
    out_specs += [pl.BlockSpec((BN, 16), lambda i: (i, 0)) for _ in range(2)]
    if has_proj:
        da = projWt.shape[1]
        in_specs += [pl.BlockSpec((ind, da), lambda i: (0, 0)),
                     pl.BlockSpec((1, da), lambda i: (0, 0))]
        args += [projWt, projb]
        out_shape.append(jax.ShapeDtypeStruct((NPAD, da), F32))
        out_specs.append(pl.BlockSpec((BN, da), lambda i: (i, 0)))
    return pl.pallas_call(
        body, grid=grid, in_specs=in_specs, out_specs=out_specs,
        out_shape=out_shape)(*args)


def _dyn_gather(row, idx):
    return lax.gather(
        row, idx[:, None],
        lax.GatherDimensionNumbers(offset_dims=(), collapsed_slice_dims=(0,),
                                   start_index_map=(0,)),
        slice_sizes=(1,), mode=lax.GatherScatterMode.PROMISE_IN_BOUNDS)


def _edge_call(hgs, ab_s, ab_d, srcp, dstp, epad, q):
    mesh = plsc.VectorSubcoreMesh(core_axis_name="c", subcore_axis_name="s")
    out_type = [
        jax.ShapeDtypeStruct((2, NG, NPAD, 128), F32),
        jax.ShapeDtypeStruct((2, NPAD, 16), F32),
        jax.ShapeDtypeStruct((epad, 16), F32),
    ]
    scratch = [
        pltpu.VMEM((C,), jnp.int32),
        pltpu.VMEM((C,), jnp.int32),
        pltpu.VMEM((C,), jnp.int32),
        pltpu.VMEM((C,), jnp.int32),
        pltpu.VMEM((C, 16), F32),
        pltpu.VMEM((C, 16), F32),
        pltpu.VMEM((C, 128), F32),
        pltpu.VMEM((C, 128), F32),
        pltpu.SemaphoreType.DMA,
        pltpu.SemaphoreType.DMA,
        pltpu.SemaphoreType.DMA,
        pltpu.VMEM_SHARED((NPAD, 128), F32),
        pltpu.VMEM_SHARED((NPAD, 16), F32),
    ]

    @functools.partial(pl.kernel, mesh=mesh, out_type=out_type,
                       scratch_types=scratch,
                       compiler_params=pltpu.CompilerParams(
                           use_tc_tiling_on_sc=False))
    def k(h0, h1, h2, h3, absr, abdr, srcr, dstr,
          accs, ssum, wg,
          idxs0, idxd0, idxs1, idxd1, Wv0, Wv1, rows0, rows1,
          sem, isem, gsem, acc_sp, s_sp):
        cid = lax.axis_index("c")
        sid = lax.axis_index("s")
        t = cid * 16 + sid
        r0 = sid * ROWS_PER_TILE

        def zw(e, _):
            Wv0[e, :] = jnp.zeros((16,), F32)
            return 0
        lax.fori_loop(0, C, zw, 0)
        for part in range(ROWS_PER_TILE // C):
            pltpu.sync_copy(Wv0, s_sp.at[pl.ds(r0 + C * part, C)])
        plsc.subcore_barrier()

        def chunk1(i, _):
            base = (t * q + i) * C
            pltpu.sync_copy(srcr.at[pl.ds(base, C)], idxs0)
            pltpu.sync_copy(dstr.at[pl.ds(base, C)], idxd0)
            pltpu.async_copy(absr.at[idxs0], Wv0, sem).wait()
            pltpu.async_copy(abdr.at[idxd0], Wv1, sem).wait()

            def edge(e, _):
                z = Wv0[e, :] + Wv1[e, :]
                Wv0[e, :] = jnp.exp(jnp.maximum(z, 0.2 * z))
                return 0
            lax.fori_loop(0, C, edge, 0)
            pltpu.sync_copy(Wv0, s_sp.at[idxd0], add=True)
            pltpu.sync_copy(Wv0, wg.at[pl.ds(base, C)])
            return 0
        lax.fori_loop(0, q, chunk1, 0)
        plsc.subcore_barrier()
        pltpu.sync_copy(s_sp.at[pl.ds(r0, ROWS_PER_TILE)],
                        ssum.at[cid, pl.ds(r0, ROWS_PER_TILE)])

        bufs = [(idxs0, idxd0, Wv0, rows0), (idxs1, idxd1, Wv1, rows1)]

        def _prefetch(i, b):
            base = (t * q + i) * C
            pltpu.async_copy(srcr.at[pl.ds(base, C)], b[0], isem)
            pltpu.async_copy(dstr.at[pl.ds(base, C)], b[1], isem)
            pltpu.async_copy(wg.at[pl.ds(base, C)], b[2], isem)

        def _prefetch_wait(i, b):
            base = (t * q + i) * C
            pltpu.make_async_copy(srcr.at[pl.ds(base, C)], b[0], isem).wait()
            pltpu.make_async_copy(dstr.at[pl.ds(base, C)], b[1], isem).wait()
            pltpu.make_async_copy(wg.at[pl.ds(base, C)], b[2], isem).wait()

        for g in range(NG):
            hg = (h0, h1, h2, h3)[g]

            def zr(e, _):
                for j in range(8):
                    rows0[e, pl.ds(16 * j, 16)] = jnp.zeros((16,), F32)
                return 0
            lax.fori_loop(0, C, zr, 0)
            for part in range(ROWS_PER_TILE // C):
                pltpu.sync_copy(rows0, acc_sp.at[pl.ds(r0 + C * part, C)])
            plsc.subcore_barrier()

            gi0 = jnp.full((16,), 2 * g, jnp.int32)
            gi1 = jnp.full((16,), 2 * g + 1, jnp.int32)

            base0 = (t * q) * C
            pltpu.sync_copy(srcr.at[pl.ds(base0, C)], idxs0)
            pltpu.sync_copy(dstr.at[pl.ds(base0, C)], idxd0)
            pltpu.sync_copy(wg.at[pl.ds(base0, C)], Wv0)
            pltpu.async_copy(hg.at[idxs0], rows0, gsem)
            _prefetch(1, bufs[1])

            def step(i, cur, nxt):
                @pl.when(i + 1 < q)
                def _():
                    _prefetch_wait(i + 1, nxt)
                pltpu.make_async_copy(hg.at[cur[0]], cur[3], gsem).wait()

                @pl.when(i + 1 < q)
                def _():
                    pltpu.async_copy(hg.at[nxt[0]], nxt[3], gsem)

                wv, rows = cur[2], cur[3]

                def edge(k2, _):
                    for e in (2 * k2, 2 * k2 + 1):
                        wrow = wv[e, :]
                        b0 = _dyn_gather(wrow, gi0)
                        b1 = _dyn_gather(wrow, gi1)
                        for j in range(4):
                            rows[e, pl.ds(16 * j, 16)] = \
                                rows[e, pl.ds(16 * j, 16)] * b0
                        for j in range(4, 8):
                            rows[e, pl.ds(16 * j, 16)] = \
                                rows[e, pl.ds(16 * j, 16)] * b1
                    return 0
                lax.fori_loop(0, C // 2, edge, 0)
                pltpu.sync_copy(rows, acc_sp.at[cur[1]], add=True)

                @pl.when(i + 2 < q)
                def _():
                    _prefetch(i + 2, cur)

            def chunk2(ip, _):
                step(2 * ip, bufs[0], bufs[1])
                step(2 * ip + 1, bufs[1], bufs[0])
                return 0
            lax.fori_loop(0, q // 2, chunk2, 0)
            plsc.subcore_barrier()
            pltpu.sync_copy(acc_sp.at[pl.ds(r0, ROWS_PER_TILE)],
                            accs.at[cid, g, pl.ds(r0, ROWS_PER_TILE)])

    accs, ssum, _ = k(hgs[0], hgs[1], hgs[2], hgs[3], ab_s, ab_d, srcp, dstp)
    return accs, ssum


def _post_call(accs, ssum, proj, bias, ln_g, ln_b, sw, concat, last):
    da = HD if concat else OC
    BN = 1280
    grid = (NPAD // BN,)

    def body(accs_ref, ssum_ref, proj_ref, bias_ref, g_ref, b_ref, sw_ref,
             out_ref):
        s = ssum_ref[0] + ssum_ref[1]
        cols = []
        for k in range(HEADS):
            g, m = k // 2, k % 2
            a = (accs_ref[0, g][:, OC * m:OC * (m + 1)]
                 + accs_ref[1, g][:, OC * m:OC * (m + 1)])
            den = s[:, k:k + 1] + 1e-16
            cols.append(a / den)
        if concat:
            xg = jnp.concatenate(cols, axis=1)
        else:
            acc = cols[0]
            for ck in cols[1:]:
                acc = acc + ck
            xg = acc / float(HEADS)
        xg = xg + bias_ref[...]
        mu = jnp.mean(xg, axis=1, keepdims=True)
        var = jnp.mean((xg - mu) * (xg - mu), axis=1, keepdims=True)
        xn = (xg - mu) * lax.rsqrt(var + 1e-5) * g_ref[...] + b_ref[...]
        res = xn + sw_ref[...] * proj_ref[...]
        if last:
            out_ref[...] = res
        else:
            out_ref[...] = jnp.where(res > 0, res, jnp.exp(res) - 1.0)

    in_specs = [
        pl.BlockSpec((2, NG, BN, 128), lambda i: (0, 0, i, 0)),
        pl.BlockSpec((2, BN, 16), lambda i: (0, i, 0)),
        pl.BlockSpec((BN, da), lambda i: (i, 0)),
        pl.BlockSpec((1, da), lambda i: (0, 0)),
        pl.BlockSpec((1, da), lambda i: (0, 0)),
        pl.BlockSpec((1, da), lambda i: (0, 0)),
        pl.BlockSpec((1, 1), lambda i: (0, 0)),
    ]
    return pl.pallas_call(
        body, grid=grid, in_specs=in_specs,
        out_specs=pl.BlockSpec((BN, da), lambda i: (i, 0)),
        out_shape=jax.ShapeDtypeStruct((NPAD, da), F32),
    )(accs, ssum, proj, bias, ln_g, ln_b, sw)


def _fusion_call(f0, f1, pw1t, pb1, pw2t, pb2, wa, wb, wgc, fb1, w2t, fb2):
    def body(f0r, f1r, pw1r, pb1r, pw2r, pb2r, war, wbr, wgr, fb1r, w2r,
             fb2r, out_ref):
        x0 = f0r[...]
        t1 = jnp.tanh(jnp.dot(x0, pw1r[...], preferred_element_type=F32)
                      + pb1r[...])
        sc = jnp.dot(t1, pw2r[...], preferred_element_type=F32) + pb2r[...]
        rid = lax.broadcasted_iota(jnp.int32, (NPAD, 1), 0)
        valid = rid < N
        scm = jnp.where(valid, sc, jnp.full_like(sc, -1e30))
        m = jnp.max(scm)
        ex = jnp.where(valid, jnp.exp(sc - m), jnp.zeros_like(sc))
        wgt = ex / jnp.sum(ex)
        gvec = jnp.sum(x0 * wgt, axis=0, keepdims=True)
        h1 = (jnp.dot(x0, war[...], preferred_element_type=F32)
              + jnp.dot(f1r[...], wbr[...], preferred_element_type=F32)
              + jnp.dot(gvec, wgr[...], preferred_element_type=F32)
              + fb1r[...])
        h1 = jnp.maximum(h1, 0.0)
        out_ref[...] = jnp.dot(h1, w2r[...],
                               preferred_element_type=F32) + fb2r[...]

    return pl.pallas_call(
        body,
        out_shape=jax.ShapeDtypeStruct((NPAD, 128), F32),
    )(f0, f1, pw1t, pb1, pw2t, pb2, wa, wb, wgc, fb1, w2t, fb2)


def _layer(h_in, p, c, srcp, dstp, epad, q, last):
    att_s = p["att_src"].reshape(1, HD)
    att_d = p["att_dst"].reshape(1, HD)
    if c["has_proj"]:
        projWt = p["proj_W"].T
        projb = p["proj_b"].reshape(1, -1)
        pre = _pre_call(h_in, p["W"], att_s, att_d, projWt, projb)
        hgs, ab_s, ab_d, proj = pre[:4], pre[4], pre[5], pre[6]
    else:
        pre = _pre_call(h_in, p["W"], att_s, att_d, None, None)
        hgs, ab_s, ab_d, proj = pre[:4], pre[4], pre[5], h_in
    accs, ssum = _edge_call(hgs, ab_s, ab_d, srcp, dstp, epad, q)
    sw = jax.nn.sigmoid(p["res_w"]).reshape(1, 1)
    return _post_call(accs, ssum, proj, p["bias"].reshape(1, -1),
                      p["ln_g"].reshape(1, -1), p["ln_b"].reshape(1, -1),
                      sw, c["concat"], last)


def kernel(x, params, edge_index):
    xp = jnp.pad(x, ((0, NPAD - N), (0, 0)))
    loop = jnp.arange(N, dtype=edge_index.dtype)
    src = jnp.concatenate([edge_index[0], loop]).astype(jnp.int32)
    dst = jnp.concatenate([edge_index[1], loop]).astype(jnp.int32)
    ne = src.shape[0]
    q = -(-ne // (NTILES * C))
    q = q + (q % 2)
    epad = NTILES * q * C
    npe = epad - ne
    pad_idx = (N + (jnp.arange(npe, dtype=jnp.int32) % 16)).astype(jnp.int32)
    srcp = jnp.concatenate([src, pad_idx])
    dstp = jnp.concatenate([dst, pad_idx])

    encs = params["encoders"]
    cfgs = [_cfgs_for(2), _cfgs_for(3)]
    state = [xp, xp]
    for li in range(3):
        for enc in range(2):
            if li >= len(cfgs[enc]):
                continue
            c = cfgs[enc][li]
            last = (li == len(cfgs[enc]) - 1)
            state[enc] = _layer(state[enc], encs[enc][li], c, srcp, dstp,
                                epad, q, last)

    pool = params["pool"]
    fu = params["fusion"]
    W1 = fu["W1"]
    out = _fusion_call(
        state[0], state[1],
        pool["W1"].T, pool["b1"].reshape(1, -1),
        pool["W2"].T, pool["b2"].reshape(1, -1),
        W1[:, :OC].T, W1[:, OC:2 * OC].T, W1[:, 2 * OC:].T,
        fu["b1"].reshape(1, -1), fu["W2"].T, fu["b2"].reshape(1, -1))
    return out[:N]


# speedup vs baseline: 42.0587x; 1.2336x over previous
import functools

import jax
import jax.numpy as jnp
from jax import lax
from jax.experimental import pallas as pl
from jax.experimental.pallas import tpu as pltpu
from jax.experimental.pallas import tpu_sc as plsc

N = 10000
NPAD = 10240
HEADS = 8
OC = 64
HD = HEADS * OC
NG = 4
C = 128
NTILES = 32
ROWS_PER_TILE = NPAD // 16
F32 = jnp.float32


def _cfgs_for(num_layers, in_ch=128, hid=64, out=64, heads=HEADS):
    cfgs = []
    cur = in_ch
    for i in range(num_layers):
        last = (i == num_layers - 1)
        oc = out if last else hid
        concat = (not last)
        da = oc * heads if concat else oc
        cfgs.append(dict(in_dim=cur, out_ch=oc, heads=heads, concat=concat,
                         dim_after=da, has_proj=(cur != da)))
        cur = da
    return cfgs


def _pre_call(xp, W, att_s, att_d, projWt, projb):
    ind = xp.shape[1]
    has_proj = projWt is not None
    BN = 1280
    grid = (NPAD // BN,)

    def body(x_ref, w_ref, as_ref, ad_ref, *rest):
        if has_proj:
            pw_ref, pb_ref = rest[0], rest[1]
            outs = rest[2:]
        else:
            outs = rest
        h0, h1, h2, h3, abs_ref, abd_ref = outs[:6]
        xb = x_ref[...]
        hb = jnp.dot(xb, w_ref[...], preferred_element_type=F32)
        acols, bcols = [], []
        for k in range(HEADS):
            hk = hb[:, OC * k:OC * (k + 1)]
            acols.append(jnp.sum(hk * as_ref[:, OC * k:OC * (k + 1)], axis=1,
                                 keepdims=True))
            bcols.append(jnp.sum(hk * ad_ref[:, OC * k:OC * (k + 1)], axis=1,
                                 keepdims=True))
        zeros8 = jnp.zeros((BN, 8), F32)
        abs_ref[...] = jnp.concatenate(acols + [zeros8], axis=1)
        abd_ref[...] = jnp.concatenate(bcols + [zeros8], axis=1)
        for g, hg in enumerate((h0, h1, h2, h3)):
            hg[...] = hb[:, 128 * g:128 * (g + 1)]
        if has_proj:
            da = pw_ref.shape[1]
            outs[6][...] = jnp.dot(xb, pw_ref[...],
                                   preferred_element_type=F32) + pb_ref[...]

    in_specs = [
        pl.BlockSpec((BN, ind), lambda i: (i, 0)),
        pl.BlockSpec((ind, HD), lambda i: (0, 0)),
        pl.BlockSpec((1, HD), lambda i: (0, 0)),
        pl.BlockSpec((1, HD), lambda i: (0, 0)),
    ]
    args = [xp, W, att_s, att_d]
    out_shape = [jax.ShapeDtypeStruct((NPAD, 128), F32) for _ in range(4)]
    out_shape += [jax.ShapeDtypeStruct((NPAD, 16), F32) for _ in range(2)]
    out_specs = [pl.BlockSpec((BN, 128), lambda i: (i, 0)) for _ in range(4)]
    out_specs += [pl.BlockSpec((BN, 16), lambda i: (i, 0)) for _ in range(2)]
    if has_proj:
        da = projWt.shape[1]
        in_specs += [pl.BlockSpec((ind, da), lambda i: (0, 0)),
                     pl.BlockSpec((1, da), lambda i: (0, 0))]
        args += [projWt, projb]
        out_shape.append(jax.ShapeDtypeStruct((NPAD, da), F32))
        out_specs.append(pl.BlockSpec((BN, da), lambda i: (i, 0)))
    return pl.pallas_call(
        body, grid=grid, in_specs=in_specs, out_specs=out_specs,
        out_shape=out_shape)(*args)


def _dyn_gather(row, idx):
    return lax.gather(
        row, idx[:, None],
        lax.GatherDimensionNumbers(offset_dims=(), collapsed_slice_dims=(0,),
                                   start_index_map=(0,)),
        slice_sizes=(1,), mode=lax.GatherScatterMode.PROMISE_IN_BOUNDS)


def _edge_call(hgs, ab_s, ab_d, srcp, dstp, epad, q):
    mesh = plsc.VectorSubcoreMesh(core_axis_name="c", subcore_axis_name="s")
    out_type = [
        jax.ShapeDtypeStruct((2, NG, NPAD, 128), F32),
        jax.ShapeDtypeStruct((2, NPAD, 16), F32),
        jax.ShapeDtypeStruct((epad, 16), F32),
    ]
    scratch = [
        pltpu.VMEM((C,), jnp.int32),
        pltpu.VMEM((C,), jnp.int32),
        pltpu.VMEM((C,), jnp.int32),
        pltpu.VMEM((C,), jnp.int32),
        pltpu.VMEM((C, 16), F32),
        pltpu.VMEM((C, 16), F32),
        pltpu.VMEM((C, 128), F32),
        pltpu.VMEM((C, 128), F32),
        pltpu.SemaphoreType.DMA,
        pltpu.SemaphoreType.DMA,
        pltpu.SemaphoreType.DMA,
        pltpu.VMEM_SHARED((NPAD, 128), F32),
        pltpu.VMEM_SHARED((NPAD, 16), F32),
    ]

    @functools.partial(pl.kernel, mesh=mesh, out_type=out_type,
                       scratch_types=scratch,
                       compiler_params=pltpu.CompilerParams(
                           use_tc_tiling_on_sc=False))
    def k(h0, h1, h2, h3, absr, abdr, srcr, dstr,
          accs, ssum, wg,
          idxs0, idxd0, idxs1, idxd1, Wv0, Wv1, rows0, rows1,
          sem, isem, gsem, acc_sp, s_sp):
        cid = lax.axis_index("c")
        sid = lax.axis_index("s")
        t = cid * 16 + sid
        r0 = sid * ROWS_PER_TILE

        def zw(e, _):
            Wv0[e, :] = jnp.zeros((16,), F32)
            return 0
        lax.fori_loop(0, C, zw, 0)
        for part in range(ROWS_PER_TILE // C):
            pltpu.sync_copy(Wv0, s_sp.at[pl.ds(r0 + C * part, C)])
        plsc.subcore_barrier()

        def chunk1(i, _):
            base = (t * q + i) * C
            pltpu.sync_copy(srcr.at[pl.ds(base, C)], idxs0)
            pltpu.sync_copy(dstr.at[pl.ds(base, C)], idxd0)
            pltpu.async_copy(absr.at[idxs0], Wv0, sem).wait()
            pltpu.async_copy(abdr.at[idxd0], Wv1, sem).wait()

            @plsc.parallel_loop(0, C, unroll=4)
            def _(e):
                z = Wv0[e, :] + Wv1[e, :]
                Wv0[e, :] = jnp.exp(jnp.maximum(z, 0.2 * z))
            pltpu.sync_copy(Wv0, s_sp.at[idxd0], add=True)
            pltpu.sync_copy(Wv0, wg.at[pl.ds(base, C)])
            return 0
        lax.fori_loop(0, q, chunk1, 0)
        plsc.subcore_barrier()
        pltpu.sync_copy(s_sp.at[pl.ds(r0, ROWS_PER_TILE)],
                        ssum.at[cid, pl.ds(r0, ROWS_PER_TILE)])

        bufs = [(idxs0, idxd0, Wv0, rows0), (idxs1, idxd1, Wv1, rows1)]

        def _prefetch(i, b):
            base = (t * q + i) * C
            pltpu.async_copy(srcr.at[pl.ds(base, C)], b[0], isem)
            pltpu.async_copy(dstr.at[pl.ds(base, C)], b[1], isem)
            pltpu.async_copy(wg.at[pl.ds(base, C)], b[2], isem)

        def _prefetch_wait(i, b):
            base = (t * q + i) * C
            pltpu.make_async_copy(srcr.at[pl.ds(base, C)], b[0], isem).wait()
            pltpu.make_async_copy(dstr.at[pl.ds(base, C)], b[1], isem).wait()
            pltpu.make_async_copy(wg.at[pl.ds(base, C)], b[2], isem).wait()

        for g in range(NG):
            hg = (h0, h1, h2, h3)[g]

            def zr(e, _):
                for j in range(8):
                    rows0[e, pl.ds(16 * j, 16)] = jnp.zeros((16,), F32)
                return 0
            lax.fori_loop(0, C, zr, 0)
            for part in range(ROWS_PER_TILE // C):
                pltpu.sync_copy(rows0, acc_sp.at[pl.ds(r0 + C * part, C)])
            plsc.subcore_barrier()

            gi0 = jnp.full((16,), 2 * g, jnp.int32)
            gi1 = jnp.full((16,), 2 * g + 1, jnp.int32)

            base0 = (t * q) * C
            pltpu.sync_copy(srcr.at[pl.ds(base0, C)], idxs0)
            pltpu.sync_copy(dstr.at[pl.ds(base0, C)], idxd0)
            pltpu.sync_copy(wg.at[pl.ds(base0, C)], Wv0)
            pltpu.async_copy(hg.at[idxs0], rows0, gsem)
            _prefetch(1, bufs[1])

            def step(i, cur, nxt):
                @pl.when(i + 1 < q)
                def _():
                    _prefetch_wait(i + 1, nxt)
                pltpu.make_async_copy(hg.at[cur[0]], cur[3], gsem).wait()

                @pl.when(i + 1 < q)
                def _():
                    pltpu.async_copy(hg.at[nxt[0]], nxt[3], gsem)

                wv, rows = cur[2], cur[3]

                @plsc.parallel_loop(0, C, unroll=4)
                def _(e):
                    wrow = wv[e, :]
                    b0 = _dyn_gather(wrow, gi0)
                    b1 = _dyn_gather(wrow, gi1)
                    for j in range(4):
                        rows[e, pl.ds(16 * j, 16)] = \
                            rows[e, pl.ds(16 * j, 16)] * b0
                    for j in range(4, 8):
                        rows[e, pl.ds(16 * j, 16)] = \
                            rows[e, pl.ds(16 * j, 16)] * b1
                pltpu.sync_copy(rows, acc_sp.at[cur[1]], add=True)

                @pl.when(i + 2 < q)
                def _():
                    _prefetch(i + 2, cur)

            def chunk2(ip, _):
                step(2 * ip, bufs[0], bufs[1])
                step(2 * ip + 1, bufs[1], bufs[0])
                return 0
            lax.fori_loop(0, q // 2, chunk2, 0)
            plsc.subcore_barrier()
            pltpu.sync_copy(acc_sp.at[pl.ds(r0, ROWS_PER_TILE)],
                            accs.at[cid, g, pl.ds(r0, ROWS_PER_TILE)])

    accs, ssum, _ = k(hgs[0], hgs[1], hgs[2], hgs[3], ab_s, ab_d, srcp, dstp)
    return accs, ssum


def _post_call(accs, ssum, proj, bias, ln_g, ln_b, sw, concat, last):
    da = HD if concat else OC
    BN = 1280
    grid = (NPAD // BN,)

    def body(accs_ref, ssum_ref, proj_ref, bias_ref, g_ref, b_ref, sw_ref,
             out_ref):
        s = ssum_ref[0] + ssum_ref[1]
        cols = []
        for k in range(HEADS):
            g, m = k // 2, k % 2
            a = (accs_ref[0, g][:, OC * m:OC * (m + 1)]
                 + accs_ref[1, g][:, OC * m:OC * (m + 1)])
            den = s[:, k:k + 1] + 1e-16
            cols.append(a / den)
        if concat:
            xg = jnp.concatenate(cols, axis=1)
        else:
            acc = cols[0]
            for ck in cols[1:]:
                acc = acc + ck
            xg = acc / float(HEADS)
        xg = xg + bias_ref[...]
        mu = jnp.mean(xg, axis=1, keepdims=True)
        var = jnp.mean((xg - mu) * (xg - mu), axis=1, keepdims=True)
        xn = (xg - mu) * lax.rsqrt(var + 1e-5) * g_ref[...] + b_ref[...]
        res = xn + sw_ref[...] * proj_ref[...]
        if last:
            out_ref[...] = res
        else:
            out_ref[...] = jnp.where(res > 0, res, jnp.exp(res) - 1.0)

    in_specs = [
        pl.BlockSpec((2, NG, BN, 128), lambda i: (0, 0, i, 0)),
        pl.BlockSpec((2, BN, 16), lambda i: (0, i, 0)),
        pl.BlockSpec((BN, da), lambda i: (i, 0)),
        pl.BlockSpec((1, da), lambda i: (0, 0)),
        pl.BlockSpec((1, da), lambda i: (0, 0)),
        pl.BlockSpec((1, da), lambda i: (0, 0)),
        pl.BlockSpec((1, 1), lambda i: (0, 0)),
    ]
    return pl.pallas_call(
        body, grid=grid, in_specs=in_specs,
        out_specs=pl.BlockSpec((BN, da), lambda i: (i, 0)),
        out_shape=jax.ShapeDtypeStruct((NPAD, da), F32),
    )(accs, ssum, proj, bias, ln_g, ln_b, sw)


def _fusion_call(f0, f1, pw1t, pb1, pw2t, pb2, wa, wb, wgc, fb1, w2t, fb2):
    def body(f0r, f1r, pw1r, pb1r, pw2r, pb2r, war, wbr, wgr, fb1r, w2r,
             fb2r, out_ref):
        x0 = f0r[...]
        t1 = jnp.tanh(jnp.dot(x0, pw1r[...], preferred_element_type=F32)
                      + pb1r[...])
        sc = jnp.dot(t1, pw2r[...], preferred_element_type=F32) + pb2r[...]
        rid = lax.broadcasted_iota(jnp.int32, (NPAD, 1), 0)
        valid = rid < N
        scm = jnp.where(valid, sc, jnp.full_like(sc, -1e30))
        m = jnp.max(scm)
        ex = jnp.where(valid, jnp.exp(sc - m), jnp.zeros_like(sc))
        wgt = ex / jnp.sum(ex)
        gvec = jnp.sum(x0 * wgt, axis=0, keepdims=True)
        h1 = (jnp.dot(x0, war[...], preferred_element_type=F32)
              + jnp.dot(f1r[...], wbr[...], preferred_element_type=F32)
              + jnp.dot(gvec, wgr[...], preferred_element_type=F32)
              + fb1r[...])
        h1 = jnp.maximum(h1, 0.0)
        out_ref[...] = jnp.dot(h1, w2r[...],
                               preferred_element_type=F32) + fb2r[...]

    return pl.pallas_call(
        body,
        out_shape=jax.ShapeDtypeStruct((NPAD, 128), F32),
    )(f0, f1, pw1t, pb1, pw2t, pb2, wa, wb, wgc, fb1, w2t, fb2)


def _layer(h_in, p, c, srcp, dstp, epad, q, last):
    att_s = p["att_src"].reshape(1, HD)
    att_d = p["att_dst"].reshape(1, HD)
    if c["has_proj"]:
        projWt = p["proj_W"].T
        projb = p["proj_b"].reshape(1, -1)
        pre = _pre_call(h_in, p["W"], att_s, att_d, projWt, projb)
        hgs, ab_s, ab_d, proj = pre[:4], pre[4], pre[5], pre[6]
    else:
        pre = _pre_call(h_in, p["W"], att_s, att_d, None, None)
        hgs, ab_s, ab_d, proj = pre[:4], pre[4], pre[5], h_in
    accs, ssum = _edge_call(hgs, ab_s, ab_d, srcp, dstp, epad, q)
    sw = jax.nn.sigmoid(p["res_w"]).reshape(1, 1)
    return _post_call(accs, ssum, proj, p["bias"].reshape(1, -1),
                      p["ln_g"].reshape(1, -1), p["ln_b"].reshape(1, -1),
                      sw, c["concat"], last)


def kernel(x, params, edge_index):
    xp = jnp.pad(x, ((0, NPAD - N), (0, 0)))
    loop = jnp.arange(N, dtype=edge_index.dtype)
    src = jnp.concatenate([edge_index[0], loop]).astype(jnp.int32)
    dst = jnp.concatenate([edge_index[1], loop]).astype(jnp.int32)
    ne = src.shape[0]
    q = -(-ne // (NTILES * C))
    q = q + (q % 2)
    epad = NTILES * q * C
    npe = epad - ne
    pad_idx = (N + (jnp.arange(npe, dtype=jnp.int32) % 16)).astype(jnp.int32)
    srcp = jnp.concatenate([src, pad_idx])
    dstp = jnp.concatenate([dst, pad_idx])

    encs = params["encoders"]
    cfgs = [_cfgs_for(2), _cfgs_for(3)]
    state = [xp, xp]
    for li in range(3):
        for enc in range(2):
            if li >= len(cfgs[enc]):
                continue
            c = cfgs[enc][li]
            last = (li == len(cfgs[enc]) - 1)
            state[enc] = _layer(state[enc], encs[enc][li], c, srcp, dstp,
                                epad, q, last)

    pool = params["pool"]
    fu = params["fusion"]
    W1 = fu["W1"]
    out = _fusion_call(
        state[0], state[1],
        pool["W1"].T, pool["b1"].reshape(1, -1),
        pool["W2"].T, pool["b2"].reshape(1, -1),
        W1[:, :OC].T, W1[:, OC:2 * OC].T, W1[:, 2 * OC:].T,
        fu["b1"].reshape(1, -1), fu["W2"].T, fu["b2"].reshape(1, -1))
    return out[:N]


# speedup vs baseline: 49.5530x; 1.1782x over previous
import functools

import jax
import jax.numpy as jnp
from jax import lax
from jax.experimental import pallas as pl
from jax.experimental.pallas import tpu as pltpu
from jax.experimental.pallas import tpu_sc as plsc

N = 10000
NPAD = 10240
HEADS = 8
OC = 64
HD = HEADS * OC
NG = 4
C = 128
NTILES = 32
ROWS_PER_TILE = NPAD // 16
F32 = jnp.float32


def _cfgs_for(num_layers, in_ch=128, hid=64, out=64, heads=HEADS):
    cfgs = []
    cur = in_ch
    for i in range(num_layers):
        last = (i == num_layers - 1)
        oc = out if last else hid
        concat = (not last)
        da = oc * heads if concat else oc
        cfgs.append(dict(in_dim=cur, out_ch=oc, heads=heads, concat=concat,
                         dim_after=da, has_proj=(cur != da)))
        cur = da
    return cfgs


def _pre_call(xp, W, att_s, att_d, projWt, projb):
    ind = xp.shape[1]
    has_proj = projWt is not None
    BN = 1280
    grid = (NPAD // BN,)

    def body(x_ref, w_ref, as_ref, ad_ref, *rest):
        if has_proj:
            pw_ref, pb_ref = rest[0], rest[1]
            outs = rest[2:]
        else:
            outs = rest
        h0, h1, h2, h3, abs_ref, abd_ref = outs[:6]
        xb = x_ref[...]
        hb = jnp.dot(xb, w_ref[...], preferred_element_type=F32)
        acols, bcols = [], []
        for k in range(HEADS):
            hk = hb[:, OC * k:OC * (k + 1)]
            acols.append(jnp.sum(hk * as_ref[:, OC * k:OC * (k + 1)], axis=1,
                                 keepdims=True))
            bcols.append(jnp.sum(hk * ad_ref[:, OC * k:OC * (k + 1)], axis=1,
                                 keepdims=True))
        zeros8 = jnp.zeros((BN, 8), F32)
        abs_ref[...] = jnp.concatenate(acols + [zeros8], axis=1)
        abd_ref[...] = jnp.concatenate(bcols + [zeros8], axis=1)
        for g, hg in enumerate((h0, h1, h2, h3)):
            hg[...] = hb[:, 128 * g:128 * (g + 1)]
        if has_proj:
            da = pw_ref.shape[1]
            outs[6][...] = jnp.dot(xb, pw_ref[...],
                                   preferred_element_type=F32) + pb_ref[...]

    in_specs = [
        pl.BlockSpec((BN, ind), lambda i: (i, 0)),
        pl.BlockSpec((ind, HD), lambda i: (0, 0)),
        pl.BlockSpec((1, HD), lambda i: (0, 0)),
        pl.BlockSpec((1, HD), lambda i: (0, 0)),
    ]
    args = [xp, W, att_s, att_d]
    out_shape = [jax.ShapeDtypeStruct((NPAD, 128), F32) for _ in range(4)]
    out_shape += [jax.ShapeDtypeStruct((NPAD, 16), F32) for _ in range(2)]
    out_specs = [pl.BlockSpec((BN, 128), lambda i: (i, 0)) for _ in range(4)]
    out_specs += [pl.BlockSpec((BN, 16), lambda i: (i, 0)) for _ in range(2)]
    if has_proj:
        da = projWt.shape[1]
        in_specs += [pl.BlockSpec((ind, da), lambda i: (0, 0)),
                     pl.BlockSpec((1, da), lambda i: (0, 0))]
        args += [projWt, projb]
        out_shape.append(jax.ShapeDtypeStruct((NPAD, da), F32))
        out_specs.append(pl.BlockSpec((BN, da), lambda i: (i, 0)))
    return pl.pallas_call(
        body, grid=grid, in_specs=in_specs, out_specs=out_specs,
        out_shape=out_shape)(*args)


def _dyn_gather(row, idx):
    return lax.gather(
        row, idx[:, None],
        lax.GatherDimensionNumbers(offset_dims=(), collapsed_slice_dims=(0,),
                                   start_index_map=(0,)),
        slice_sizes=(1,), mode=lax.GatherScatterMode.PROMISE_IN_BOUNDS)


def _edge_call(hgs, ab_s, ab_d, srcp, dstp, epad, q):
    mesh = plsc.VectorSubcoreMesh(core_axis_name="c", subcore_axis_name="s")
    out_type = [
        jax.ShapeDtypeStruct((2, NG, NPAD, 128), F32),
        jax.ShapeDtypeStruct((2, NPAD, 16), F32),
        jax.ShapeDtypeStruct((epad, 16), F32),
    ]
    scratch = [
        pltpu.VMEM((C,), jnp.int32),
        pltpu.VMEM((C,), jnp.int32),
        pltpu.VMEM((C,), jnp.int32),
        pltpu.VMEM((C,), jnp.int32),
        pltpu.VMEM((C, 16), F32),
        pltpu.VMEM((C, 16), F32),
        pltpu.VMEM((C, 128), F32),
        pltpu.VMEM((C, 128), F32),
        pltpu.SemaphoreType.DMA,
        pltpu.SemaphoreType.DMA,
        pltpu.SemaphoreType.DMA,
        pltpu.SemaphoreType.DMA,
        pltpu.SemaphoreType.DMA,
        pltpu.VMEM_SHARED((NPAD, 128), F32),
        pltpu.VMEM_SHARED((NPAD, 16), F32),
    ]

    @functools.partial(pl.kernel, mesh=mesh, out_type=out_type,
                       scratch_types=scratch,
                       compiler_params=pltpu.CompilerParams(
                           use_tc_tiling_on_sc=False))
    def k(h0, h1, h2, h3, absr, abdr, srcr, dstr,
          accs, ssum, wg,
          idxs0, idxd0, idxs1, idxd1, Wv0, Wv1, rows0, rows1,
          sem, isem, gsem, ssem, dsem, acc_sp, s_sp):
        cid = lax.axis_index("c")
        sid = lax.axis_index("s")
        t = cid * 16 + sid
        r0 = sid * ROWS_PER_TILE

        def zw(e, _):
            Wv0[e, :] = jnp.zeros((16,), F32)
            return 0
        lax.fori_loop(0, C, zw, 0)
        for part in range(ROWS_PER_TILE // C):
            pltpu.sync_copy(Wv0, s_sp.at[pl.ds(r0 + C * part, C)])
        plsc.subcore_barrier()

        def chunk1(i, _):
            base = (t * q + i) * C
            pltpu.sync_copy(srcr.at[pl.ds(base, C)], idxs0)
            pltpu.sync_copy(dstr.at[pl.ds(base, C)], idxd0)
            pltpu.async_copy(absr.at[idxs0], Wv0, sem).wait()
            pltpu.async_copy(abdr.at[idxd0], Wv1, sem).wait()

            @plsc.parallel_loop(0, C, unroll=4)
            def _(e):
                z = Wv0[e, :] + Wv1[e, :]
                Wv0[e, :] = jnp.exp(jnp.maximum(z, 0.2 * z))
            pltpu.sync_copy(Wv0, s_sp.at[idxd0], add=True)
            pltpu.sync_copy(Wv0, wg.at[pl.ds(base, C)])
            return 0
        lax.fori_loop(0, q, chunk1, 0)
        plsc.subcore_barrier()
        pltpu.sync_copy(s_sp.at[pl.ds(r0, ROWS_PER_TILE)],
                        ssum.at[cid, pl.ds(r0, ROWS_PER_TILE)])

        bufs = [(idxs0, idxd0, Wv0, rows0), (idxs1, idxd1, Wv1, rows1)]

        def _prefetch(i, b):
            base = (t * q + i) * C
            pltpu.async_copy(srcr.at[pl.ds(base, C)], b[0], isem)
            pltpu.async_copy(wg.at[pl.ds(base, C)], b[2], isem)

        def _prefetch_wait(i, b):
            base = (t * q + i) * C
            pltpu.make_async_copy(srcr.at[pl.ds(base, C)], b[0], isem).wait()
            pltpu.make_async_copy(wg.at[pl.ds(base, C)], b[2], isem).wait()

        for g in range(NG):
            hg = (h0, h1, h2, h3)[g]

            def zr(e, _):
                for j in range(8):
                    rows0[e, pl.ds(16 * j, 16)] = jnp.zeros((16,), F32)
                return 0
            lax.fori_loop(0, C, zr, 0)
            for part in range(ROWS_PER_TILE // C):
                pltpu.sync_copy(rows0, acc_sp.at[pl.ds(r0 + C * part, C)])
            plsc.subcore_barrier()

            gi0 = jnp.full((16,), 2 * g, jnp.int32)
            gi1 = jnp.full((16,), 2 * g + 1, jnp.int32)

            base0 = (t * q) * C
            pltpu.sync_copy(srcr.at[pl.ds(base0, C)], idxs0)
            pltpu.sync_copy(wg.at[pl.ds(base0, C)], Wv0)
            pltpu.async_copy(dstr.at[pl.ds(base0, C)], idxd0, dsem)
            pltpu.async_copy(hg.at[idxs0], rows0, gsem)
            _prefetch(1, bufs[1])

            def step(i, cur, nxt):
                @pl.when(i + 1 < q)
                def _():
                    _prefetch_wait(i + 1, nxt)

                @pl.when(i >= 1)
                def _():
                    pltpu.make_async_copy(nxt[3], acc_sp.at[nxt[1]],
                                          ssem).wait()

                @pl.when(i + 1 < q)
                def _():
                    base = (t * q + i + 1) * C
                    pltpu.async_copy(dstr.at[pl.ds(base, C)], nxt[1], dsem)
                pltpu.make_async_copy(hg.at[cur[0]], cur[3], gsem).wait()

                @pl.when(i + 1 < q)
                def _():
                    pltpu.async_copy(hg.at[nxt[0]], nxt[3], gsem)

                wv, rows = cur[2], cur[3]

                @plsc.parallel_loop(0, C, unroll=4)
                def _(e):
                    wrow = wv[e, :]
                    b0 = _dyn_gather(wrow, gi0)
                    b1 = _dyn_gather(wrow, gi1)
                    for j in range(4):
                        rows[e, pl.ds(16 * j, 16)] = \
                            rows[e, pl.ds(16 * j, 16)] * b0
                    for j in range(4, 8):
                        rows[e, pl.ds(16 * j, 16)] = \
                            rows[e, pl.ds(16 * j, 16)] * b1
                base_i = (t * q + i) * C
                pltpu.make_async_copy(dstr.at[pl.ds(base_i, C)], cur[1],
                                      dsem).wait()
                pltpu.async_copy(rows, acc_sp.at[cur[1]], ssem, add=True)

                @pl.when(i + 2 < q)
                def _():
                    _prefetch(i + 2, cur)

            def chunk2(ip, _):
                step(2 * ip, bufs[0], bufs[1])
                step(2 * ip + 1, bufs[1], bufs[0])
                return 0
            lax.fori_loop(0, q // 2, chunk2, 0)
            pltpu.make_async_copy(bufs[(q - 1) % 2][3],
                                  acc_sp.at[bufs[(q - 1) % 2][1]],
                                  ssem).wait()
            plsc.subcore_barrier()
            pltpu.sync_copy(acc_sp.at[pl.ds(r0, ROWS_PER_TILE)],
                            accs.at[cid, g, pl.ds(r0, ROWS_PER_TILE)])

    accs, ssum, _ = k(hgs[0], hgs[1], hgs[2], hgs[3], ab_s, ab_d, srcp, dstp)
    return accs, ssum


def _post_call(accs, ssum, proj, bias, ln_g, ln_b, sw, concat, last):
    da = HD if concat else OC
    BN = 1280
    grid = (NPAD // BN,)

    def body(accs_ref, ssum_ref, proj_ref, bias_ref, g_ref, b_ref, sw_ref,
             out_ref):
        s = ssum_ref[0] + ssum_ref[1]
        cols = []
        for k in range(HEADS):
            g, m = k // 2, k % 2
            a = (accs_ref[0, g][:, OC * m:OC * (m + 1)]
                 + accs_ref[1, g][:, OC * m:OC * (m + 1)])
            den = s[:, k:k + 1] + 1e-16
            cols.append(a / den)
        if concat:
            xg = jnp.concatenate(cols, axis=1)
        else:
            acc = cols[0]
            for ck in cols[1:]:
                acc = acc + ck
            xg = acc / float(HEADS)
        xg = xg + bias_ref[...]
        mu = jnp.mean(xg, axis=1, keepdims=True)
        var = jnp.mean((xg - mu) * (xg - mu), axis=1, keepdims=True)
        xn = (xg - mu) * lax.rsqrt(var + 1e-5) * g_ref[...] + b_ref[...]
        res = xn + sw_ref[...] * proj_ref[...]
        if last:
            out_ref[...] = res
        else:
            out_ref[...] = jnp.where(res > 0, res, jnp.exp(res) - 1.0)

    in_specs = [
        pl.BlockSpec((2, NG, BN, 128), lambda i: (0, 0, i, 0)),
        pl.BlockSpec((2, BN, 16), lambda i: (0, i, 0)),
        pl.BlockSpec((BN, da), lambda i: (i, 0)),
        pl.BlockSpec((1, da), lambda i: (0, 0)),
        pl.BlockSpec((1, da), lambda i: (0, 0)),
        pl.BlockSpec((1, da), lambda i: (0, 0)),
        pl.BlockSpec((1, 1), lambda i: (0, 0)),
    ]
    return pl.pallas_call(
        body, grid=grid, in_specs=in_specs,
        out_specs=pl.BlockSpec((BN, da), lambda i: (i, 0)),
        out_shape=jax.ShapeDtypeStruct((NPAD, da), F32),
    )(accs, ssum, proj, bias, ln_g, ln_b, sw)


def _fusion_call(f0, f1, pw1t, pb1, pw2t, pb2, wa, wb, wgc, fb1, w2t, fb2):
    def body(f0r, f1r, pw1r, pb1r, pw2r, pb2r, war, wbr, wgr, fb1r, w2r,
             fb2r, out_ref):
        x0 = f0r[...]
        t1 = jnp.tanh(jnp.dot(x0, pw1r[...], preferred_element_type=F32)
                      + pb1r[...])
        sc = jnp.dot(t1, pw2r[...], preferred_element_type=F32) + pb2r[...]
        rid = lax.broadcasted_iota(jnp.int32, (NPAD, 1), 0)
        valid = rid < N
        scm = jnp.where(valid, sc, jnp.full_like(sc, -1e30))
        m = jnp.max(scm)
        ex = jnp.where(valid, jnp.exp(sc - m), jnp.zeros_like(sc))
        wgt = ex / jnp.sum(ex)
        gvec = jnp.sum(x0 * wgt, axis=0, keepdims=True)
        h1 = (jnp.dot(x0, war[...], preferred_element_type=F32)
              + jnp.dot(f1r[...], wbr[...], preferred_element_type=F32)
              + jnp.dot(gvec, wgr[...], preferred_element_type=F32)
              + fb1r[...])
        h1 = jnp.maximum(h1, 0.0)
        out_ref[...] = jnp.dot(h1, w2r[...],
                               preferred_element_type=F32) + fb2r[...]

    return pl.pallas_call(
        body,
        out_shape=jax.ShapeDtypeStruct((NPAD, 128), F32),
    )(f0, f1, pw1t, pb1, pw2t, pb2, wa, wb, wgc, fb1, w2t, fb2)


def _layer(h_in, p, c, srcp, dstp, epad, q, last):
    att_s = p["att_src"].reshape(1, HD)
    att_d = p["att_dst"].reshape(1, HD)
    if c["has_proj"]:
        projWt = p["proj_W"].T
        projb = p["proj_b"].reshape(1, -1)
        pre = _pre_call(h_in, p["W"], att_s, att_d, projWt, projb)
        hgs, ab_s, ab_d, proj = pre[:4], pre[4], pre[5], pre[6]
    else:
        pre = _pre_call(h_in, p["W"], att_s, att_d, None, None)
        hgs, ab_s, ab_d, proj = pre[:4], pre[4], pre[5], h_in
    accs, ssum = _edge_call(hgs, ab_s, ab_d, srcp, dstp, epad, q)
    sw = jax.nn.sigmoid(p["res_w"]).reshape(1, 1)
    return _post_call(accs, ssum, proj, p["bias"].reshape(1, -1),
                      p["ln_g"].reshape(1, -1), p["ln_b"].reshape(1, -1),
                      sw, c["concat"], last)


def kernel(x, params, edge_index):
    xp = jnp.pad(x, ((0, NPAD - N), (0, 0)))
    loop = jnp.arange(N, dtype=edge_index.dtype)
    src = jnp.concatenate([edge_index[0], loop]).astype(jnp.int32)
    dst = jnp.concatenate([edge_index[1], loop]).astype(jnp.int32)
    ne = src.shape[0]
    q = -(-ne // (NTILES * C))
    q = q + (q % 2)
    epad = NTILES * q * C
    npe = epad - ne
    pad_idx = (N + (jnp.arange(npe, dtype=jnp.int32) % 16)).astype(jnp.int32)
    srcp = jnp.concatenate([src, pad_idx])
    dstp = jnp.concatenate([dst, pad_idx])

    encs = params["encoders"]
    cfgs = [_cfgs_for(2), _cfgs_for(3)]
    state = [xp, xp]
    for li in range(3):
        for enc in range(2):
            if li >= len(cfgs[enc]):
                continue
            c = cfgs[enc][li]
            last = (li == len(cfgs[enc]) - 1)
            state[enc] = _layer(state[enc], encs[enc][li], c, srcp, dstp,
                                epad, q, last)

    pool = params["pool"]
    fu = params["fusion"]
    W1 = fu["W1"]
    out = _fusion_call(
        state[0], state[1],
        pool["W1"].T, pool["b1"].reshape(1, -1),
        pool["W2"].T, pool["b2"].reshape(1, -1),
        W1[:, :OC].T, W1[:, OC:2 * OC].T, W1[:, 2 * OC:].T,
        fu["b1"].reshape(1, -1), fu["W2"].T, fu["b2"].reshape(1, -1))
    return out[:N]


# speedup vs baseline: 54.9673x; 1.1093x over previous
import functools

import jax
import jax.numpy as jnp
from jax import lax
from jax.experimental import pallas as pl
from jax.experimental.pallas import tpu as pltpu
from jax.experimental.pallas import tpu_sc as plsc

N = 10000
NPAD = 10240
HEADS = 8
OC = 64
HD = HEADS * OC
NG = 4
C = 128
C1 = 256
NTILES = 32
ROWS_PER_TILE = NPAD // 16
F32 = jnp.float32


def _cfgs_for(num_layers, in_ch=128, hid=64, out=64, heads=HEADS):
    cfgs = []
    cur = in_ch
    for i in range(num_layers):
        last = (i == num_layers - 1)
        oc = out if last else hid
        concat = (not last)
        da = oc * heads if concat else oc
        cfgs.append(dict(in_dim=cur, out_ch=oc, heads=heads, concat=concat,
                         dim_after=da, has_proj=(cur != da)))
        cur = da
    return cfgs


def _pre_call(xp, W, att_s, att_d, projWt, projb):
    ind = xp.shape[1]
    has_proj = projWt is not None
    BN = 1280
    grid = (NPAD // BN,)

    def body(x_ref, w_ref, as_ref, ad_ref, *rest):
        if has_proj:
            pw_ref, pb_ref = rest[0], rest[1]
            outs = rest[2:]
        else:
            outs = rest
        h0, h1, h2, h3, abs_ref, abd_ref = outs[:6]
        xb = x_ref[...]
        hb = jnp.dot(xb, w_ref[...], preferred_element_type=F32)
        acols, bcols = [], []
        for k in range(HEADS):
            hk = hb[:, OC * k:OC * (k + 1)]
            acols.append(jnp.sum(hk * as_ref[:, OC * k:OC * (k + 1)], axis=1,
                                 keepdims=True))
            bcols.append(jnp.sum(hk * ad_ref[:, OC * k:OC * (k + 1)], axis=1,
                                 keepdims=True))
        zeros8 = jnp.zeros((BN, 8), F32)
        abs_ref[...] = jnp.concatenate(acols + [zeros8], axis=1)
        abd_ref[...] = jnp.concatenate(bcols + [zeros8], axis=1)
        for g, hg in enumerate((h0, h1, h2, h3)):
            hg[...] = hb[:, 128 * g:128 * (g + 1)]
        if has_proj:
            da = pw_ref.shape[1]
            outs[6][...] = jnp.dot(xb, pw_ref[...],
                                   preferred_element_type=F32) + pb_ref[...]

    in_specs = [
        pl.BlockSpec((BN, ind), lambda i: (i, 0)),
        pl.BlockSpec((ind, HD), lambda i: (0, 0)),
        pl.BlockSpec((1, HD), lambda i: (0, 0)),
        pl.BlockSpec((1, HD), lambda i: (0, 0)),
    ]
    args = [xp, W, att_s, att_d]
    out_shape = [jax.ShapeDtypeStruct((NPAD, 128), F32) for _ in range(4)]
    out_shape += [jax.ShapeDtypeStruct((NPAD, 16), F32) for _ in range(2)]
    out_specs = [pl.BlockSpec((BN, 128), lambda i: (i, 0)) for _ in range(4)]
    out_specs += [pl.BlockSpec((BN, 16), lambda i: (i, 0)) for _ in range(2)]
    if has_proj:
        da = projWt.shape[1]
        in_specs += [pl.BlockSpec((ind, da), lambda i: (0, 0)),
                     pl.BlockSpec((1, da), lambda i: (0, 0))]
        args += [projWt, projb]
        out_shape.append(jax.ShapeDtypeStruct((NPAD, da), F32))
        out_specs.append(pl.BlockSpec((BN, da), lambda i: (i, 0)))
    return pl.pallas_call(
        body, grid=grid, in_specs=in_specs, out_specs=out_specs,
        out_shape=out_shape)(*args)


def _dyn_gather(row, idx):
    return lax.gather(
        row, idx[:, None],
        lax.GatherDimensionNumbers(offset_dims=(), collapsed_slice_dims=(0,),
                                   start_index_map=(0,)),
        slice_sizes=(1,), mode=lax.GatherScatterMode.PROMISE_IN_BOUNDS)


def _weights_call(ab_s, ab_d, srcp, dstp, epad, q1):
    mesh = plsc.VectorSubcoreMesh(core_axis_name="c", subcore_axis_name="s")
    out_type = [
        jax.ShapeDtypeStruct((2, NPAD, 16), F32),
        jax.ShapeDtypeStruct((epad, 16), F32),
    ]
    scratch = [
        pltpu.VMEM((C1,), jnp.int32),
        pltpu.VMEM((C1,), jnp.int32),
        pltpu.VMEM((C1,), jnp.int32),
        pltpu.VMEM((C1,), jnp.int32),
        pltpu.VMEM((C1, 16), F32),
        pltpu.VMEM((C1, 16), F32),
        pltpu.VMEM((C1, 16), F32),
        pltpu.VMEM((C1, 16), F32),
        pltpu.SemaphoreType.DMA,
        pltpu.SemaphoreType.DMA,
        pltpu.SemaphoreType.DMA,
        pltpu.SemaphoreType.DMA,
        pltpu.SemaphoreType.DMA,
        pltpu.VMEM_SHARED((NPAD, 16), F32),
    ]

    @functools.partial(pl.kernel, mesh=mesh, out_type=out_type,
                       scratch_types=scratch,
                       compiler_params=pltpu.CompilerParams(
                           use_tc_tiling_on_sc=False))
    def k(absr, abdr, srcr, dstr, ssum, wg,
          is0, id0, is1, id1, S0, S1, D0, D1,
          isem, gsem, ssem, dsem, wsem, s_sp):
        cid = lax.axis_index("c")
        sid = lax.axis_index("s")
        t = cid * 16 + sid
        r0 = sid * ROWS_PER_TILE

        @plsc.parallel_loop(0, C1, unroll=4)
        def _(e):
            S0[e, :] = jnp.zeros((16,), F32)
        off = 0
        while off < ROWS_PER_TILE:
            n = min(C1, ROWS_PER_TILE - off)
            pltpu.sync_copy(S0.at[pl.ds(0, n)], s_sp.at[pl.ds(r0 + off, n)])
            off += n
        plsc.subcore_barrier()

        bufs = [(is0, id0, S0, D0), (is1, id1, S1, D1)]

        base0 = (t * q1) * C1
        pltpu.sync_copy(srcr.at[pl.ds(base0, C1)], is0)
        pltpu.sync_copy(dstr.at[pl.ds(base0, C1)], id0)
        pltpu.async_copy(absr.at[is0], S0, gsem)
        pltpu.async_copy(abdr.at[id0], D0, gsem)
        pltpu.async_copy(srcr.at[pl.ds(base0 + C1, C1)], is1, isem)

        def step(i, cur, nxt):
            @pl.when(i >= 1)
            def _():
                basep = (t * q1 + i - 1) * C1
                pltpu.make_async_copy(nxt[2], s_sp.at[nxt[1]], ssem).wait()
                pltpu.make_async_copy(nxt[2], wg.at[pl.ds(basep, C1)],
                                      wsem).wait()

            @pl.when(i + 1 < q1)
            def _():
                base = (t * q1 + i + 1) * C1
                pltpu.make_async_copy(srcr.at[pl.ds(base, C1)], nxt[0],
                                      isem).wait()
                pltpu.async_copy(dstr.at[pl.ds(base, C1)], nxt[1], dsem)
                pltpu.async_copy(absr.at[nxt[0]], nxt[2], gsem)

            pltpu.make_async_copy(absr.at[cur[0]], cur[2], gsem).wait()
            pltpu.make_async_copy(abdr.at[cur[1]], cur[3], gsem).wait()

            S, D = cur[2], cur[3]

            @plsc.parallel_loop(0, C1, unroll=4)
            def _(e):
                z = S[e, :] + D[e, :]
                S[e, :] = jnp.exp(jnp.maximum(z, 0.2 * z))

            @pl.when(i + 1 < q1)
            def _():
                base = (t * q1 + i + 1) * C1
                pltpu.make_async_copy(dstr.at[pl.ds(base, C1)], nxt[1],
                                      dsem).wait()
                pltpu.async_copy(abdr.at[nxt[1]], nxt[3], gsem)

            base_i = (t * q1 + i) * C1
            pltpu.async_copy(S, s_sp.at[cur[1]], ssem, add=True)
            pltpu.async_copy(S, wg.at[pl.ds(base_i, C1)], wsem)

            @pl.when(i + 2 < q1)
            def _():
                base = (t * q1 + i + 2) * C1
                pltpu.async_copy(srcr.at[pl.ds(base, C1)], cur[0], isem)

        def loop(ip, _):
            step(2 * ip, bufs[0], bufs[1])
            step(2 * ip + 1, bufs[1], bufs[0])
            return 0
        lax.fori_loop(0, q1 // 2, loop, 0)
        last = bufs[(q1 - 1) % 2]
        basel = (t * q1 + q1 - 1) * C1
        pltpu.make_async_copy(last[2], s_sp.at[last[1]], ssem).wait()
        pltpu.make_async_copy(last[2], wg.at[pl.ds(basel, C1)], wsem).wait()
        plsc.subcore_barrier()
        pltpu.sync_copy(s_sp.at[pl.ds(r0, ROWS_PER_TILE)],
                        ssum.at[cid, pl.ds(r0, ROWS_PER_TILE)])

    return k(ab_s, ab_d, srcp, dstp)


def _agg_call(hgs, wg, srcp, dstp, epad, q):
    mesh = plsc.VectorSubcoreMesh(core_axis_name="c", subcore_axis_name="s")
    out_type = [
        jax.ShapeDtypeStruct((2, NG, NPAD, 128), F32),
    ]
    scratch = [
        pltpu.VMEM((C,), jnp.int32),
        pltpu.VMEM((C,), jnp.int32),
        pltpu.VMEM((C,), jnp.int32),
        pltpu.VMEM((C,), jnp.int32),
        pltpu.VMEM((C, 16), F32),
        pltpu.VMEM((C, 16), F32),
        pltpu.VMEM((C, 128), F32),
        pltpu.VMEM((C, 128), F32),
        pltpu.SemaphoreType.DMA,
        pltpu.SemaphoreType.DMA,
        pltpu.SemaphoreType.DMA,
        pltpu.SemaphoreType.DMA,
        pltpu.VMEM_SHARED((NPAD, 128), F32),
    ]

    @functools.partial(pl.kernel, mesh=mesh, out_type=out_type,
                       scratch_types=scratch,
                       compiler_params=pltpu.CompilerParams(
                           use_tc_tiling_on_sc=False))
    def k(h0, h1, h2, h3, wg, srcr, dstr,
          accs,
          idxs0, idxd0, idxs1, idxd1, Wv0, Wv1, rows0, rows1,
          isem, gsem, ssem, dsem, acc_sp):
        cid = lax.axis_index("c")
        sid = lax.axis_index("s")
        t = cid * 16 + sid
        r0 = sid * ROWS_PER_TILE

        bufs = [(idxs0, idxd0, Wv0, rows0), (idxs1, idxd1, Wv1, rows1)]

        def _prefetch(i, b):
            base = (t * q + i) * C
            pltpu.async_copy(srcr.at[pl.ds(base, C)], b[0], isem)
            pltpu.async_copy(wg.at[pl.ds(base, C)], b[2], isem)

        def _prefetch_wait(i, b):
            base = (t * q + i) * C
            pltpu.make_async_copy(srcr.at[pl.ds(base, C)], b[0], isem).wait()
            pltpu.make_async_copy(wg.at[pl.ds(base, C)], b[2], isem).wait()

        for g in range(NG):
            hg = (h0, h1, h2, h3)[g]

            def zr(e, _):
                for j in range(8):
                    rows0[e, pl.ds(16 * j, 16)] = jnp.zeros((16,), F32)
                return 0
            lax.fori_loop(0, C, zr, 0)
            for part in range(ROWS_PER_TILE // C):
                pltpu.sync_copy(rows0, acc_sp.at[pl.ds(r0 + C * part, C)])
            plsc.subcore_barrier()

            gi0 = jnp.full((16,), 2 * g, jnp.int32)
            gi1 = jnp.full((16,), 2 * g + 1, jnp.int32)

            base0 = (t * q) * C
            pltpu.sync_copy(srcr.at[pl.ds(base0, C)], idxs0)
            pltpu.sync_copy(wg.at[pl.ds(base0, C)], Wv0)
            pltpu.async_copy(dstr.at[pl.ds(base0, C)], idxd0, dsem)
            pltpu.async_copy(hg.at[idxs0], rows0, gsem)
            _prefetch(1, bufs[1])

            def step(i, cur, nxt):
                @pl.when(i + 1 < q)
                def _():
                    _prefetch_wait(i + 1, nxt)

                @pl.when(i >= 1)
                def _():
                    pltpu.make_async_copy(nxt[3], acc_sp.at[nxt[1]],
                                          ssem).wait()

                @pl.when(i + 1 < q)
                def _():
                    base = (t * q + i + 1) * C
                    pltpu.async_copy(dstr.at[pl.ds(base, C)], nxt[1], dsem)
                pltpu.make_async_copy(hg.at[cur[0]], cur[3], gsem).wait()

                @pl.when(i + 1 < q)
                def _():
                    pltpu.async_copy(hg.at[nxt[0]], nxt[3], gsem)

                wv, rows = cur[2], cur[3]

                @plsc.parallel_loop(0, C, unroll=4)
                def _(e):
                    wrow = wv[e, :]
                    b0 = _dyn_gather(wrow, gi0)
                    b1 = _dyn_gather(wrow, gi1)
                    for j in range(4):
                        rows[e, pl.ds(16 * j, 16)] = \
                            rows[e, pl.ds(16 * j, 16)] * b0
                    for j in range(4, 8):
                        rows[e, pl.ds(16 * j, 16)] = \
                            rows[e, pl.ds(16 * j, 16)] * b1
                base_i = (t * q + i) * C
                pltpu.make_async_copy(dstr.at[pl.ds(base_i, C)], cur[1],
                                      dsem).wait()
                pltpu.async_copy(rows, acc_sp.at[cur[1]], ssem, add=True)

                @pl.when(i + 2 < q)
                def _():
                    _prefetch(i + 2, cur)

            def chunk2(ip, _):
                step(2 * ip, bufs[0], bufs[1])
                step(2 * ip + 1, bufs[1], bufs[0])
                return 0
            lax.fori_loop(0, q // 2, chunk2, 0)
            pltpu.make_async_copy(bufs[(q - 1) % 2][3],
                                  acc_sp.at[bufs[(q - 1) % 2][1]],
                                  ssem).wait()
            plsc.subcore_barrier()
            pltpu.sync_copy(acc_sp.at[pl.ds(r0, ROWS_PER_TILE)],
                            accs.at[cid, g, pl.ds(r0, ROWS_PER_TILE)])

    return k(hgs[0], hgs[1], hgs[2], hgs[3], wg, srcp, dstp)[0]


def _edge_call(hgs, ab_s, ab_d, srcp, dstp, epad, q):
    ssum, wg = _weights_call(ab_s, ab_d, srcp, dstp, epad, q * C // C1)
    accs = _agg_call(hgs, wg, srcp, dstp, epad, q)
    return accs, ssum


def _post_call(accs, ssum, proj, bias, ln_g, ln_b, sw, concat, last):
    da = HD if concat else OC
    BN = 1280
    grid = (NPAD // BN,)

    def body(accs_ref, ssum_ref, proj_ref, bias_ref, g_ref, b_ref, sw_ref,
             out_ref):
        s = ssum_ref[0] + ssum_ref[1]
        cols = []
        for k in range(HEADS):
            g, m = k // 2, k % 2
            a = (accs_ref[0, g][:, OC * m:OC * (m + 1)]
                 + accs_ref[1, g][:, OC * m:OC * (m + 1)])
            den = s[:, k:k + 1] + 1e-16
            cols.append(a / den)
        if concat:
            xg = jnp.concatenate(cols, axis=1)
        else:
            acc = cols[0]
            for ck in cols[1:]:
                acc = acc + ck
            xg = acc / float(HEADS)
        xg = xg + bias_ref[...]
        mu = jnp.mean(xg, axis=1, keepdims=True)
        var = jnp.mean((xg - mu) * (xg - mu), axis=1, keepdims=True)
        xn = (xg - mu) * lax.rsqrt(var + 1e-5) * g_ref[...] + b_ref[...]
        res = xn + sw_ref[...] * proj_ref[...]
        if last:
            out_ref[...] = res
        else:
            out_ref[...] = jnp.where(res > 0, res, jnp.exp(res) - 1.0)

    in_specs = [
        pl.BlockSpec((2, NG, BN, 128), lambda i: (0, 0, i, 0)),
        pl.BlockSpec((2, BN, 16), lambda i: (0, i, 0)),
        pl.BlockSpec((BN, da), lambda i: (i, 0)),
        pl.BlockSpec((1, da), lambda i: (0, 0)),
        pl.BlockSpec((1, da), lambda i: (0, 0)),
        pl.BlockSpec((1, da), lambda i: (0, 0)),
        pl.BlockSpec((1, 1), lambda i: (0, 0)),
    ]
    return pl.pallas_call(
        body, grid=grid, in_specs=in_specs,
        out_specs=pl.BlockSpec((BN, da), lambda i: (i, 0)),
        out_shape=jax.ShapeDtypeStruct((NPAD, da), F32),
    )(accs, ssum, proj, bias, ln_g, ln_b, sw)


def _fusion_call(f0, f1, pw1t, pb1, pw2t, pb2, wa, wb, wgc, fb1, w2t, fb2):
    def body(f0r, f1r, pw1r, pb1r, pw2r, pb2r, war, wbr, wgr, fb1r, w2r,
             fb2r, out_ref):
        x0 = f0r[...]
        t1 = jnp.tanh(jnp.dot(x0, pw1r[...], preferred_element_type=F32)
                      + pb1r[...])
        sc = jnp.dot(t1, pw2r[...], preferred_element_type=F32) + pb2r[...]
        rid = lax.broadcasted_iota(jnp.int32, (NPAD, 1), 0)
        valid = rid < N
        scm = jnp.where(valid, sc, jnp.full_like(sc, -1e30))
        m = jnp.max(scm)
        ex = jnp.where(valid, jnp.exp(sc - m), jnp.zeros_like(sc))
        wgt = ex / jnp.sum(ex)
        gvec = jnp.sum(x0 * wgt, axis=0, keepdims=True)
        h1 = (jnp.dot(x0, war[...], preferred_element_type=F32)
              + jnp.dot(f1r[...], wbr[...], preferred_element_type=F32)
              + jnp.dot(gvec, wgr[...], preferred_element_type=F32)
              + fb1r[...])
        h1 = jnp.maximum(h1, 0.0)
        out_ref[...] = jnp.dot(h1, w2r[...],
                               preferred_element_type=F32) + fb2r[...]

    return pl.pallas_call(
        body,
        out_shape=jax.ShapeDtypeStruct((NPAD, 128), F32),
    )(f0, f1, pw1t, pb1, pw2t, pb2, wa, wb, wgc, fb1, w2t, fb2)


def _layer(h_in, p, c, srcp, dstp, epad, q, last):
    att_s = p["att_src"].reshape(1, HD)
    att_d = p["att_dst"].reshape(1, HD)
    if c["has_proj"]:
        projWt = p["proj_W"].T
        projb = p["proj_b"].reshape(1, -1)
        pre = _pre_call(h_in, p["W"], att_s, att_d, projWt, projb)
        hgs, ab_s, ab_d, proj = pre[:4], pre[4], pre[5], pre[6]
    else:
        pre = _pre_call(h_in, p["W"], att_s, att_d, None, None)
        hgs, ab_s, ab_d, proj = pre[:4], pre[4], pre[5], h_in
    accs, ssum = _edge_call(hgs, ab_s, ab_d, srcp, dstp, epad, q)
    sw = jax.nn.sigmoid(p["res_w"]).reshape(1, 1)
    return _post_call(accs, ssum, proj, p["bias"].reshape(1, -1),
                      p["ln_g"].reshape(1, -1), p["ln_b"].reshape(1, -1),
                      sw, c["concat"], last)


def kernel(x, params, edge_index):
    xp = jnp.pad(x, ((0, NPAD - N), (0, 0)))
    loop = jnp.arange(N, dtype=edge_index.dtype)
    src = jnp.concatenate([edge_index[0], loop]).astype(jnp.int32)
    dst = jnp.concatenate([edge_index[1], loop]).astype(jnp.int32)
    ne = src.shape[0]
    q = -(-ne // (NTILES * C))
    q = q + (-q) % 4
    epad = NTILES * q * C
    npe = epad - ne
    pad_idx = (N + (jnp.arange(npe, dtype=jnp.int32) % 16)).astype(jnp.int32)
    srcp = jnp.concatenate([src, pad_idx])
    dstp = jnp.concatenate([dst, pad_idx])

    encs = params["encoders"]
    cfgs = [_cfgs_for(2), _cfgs_for(3)]
    state = [xp, xp]
    for li in range(3):
        for enc in range(2):
            if li >= len(cfgs[enc]):
                continue
            c = cfgs[enc][li]
            last = (li == len(cfgs[enc]) - 1)
            state[enc] = _layer(state[enc], encs[enc][li], c, srcp, dstp,
                                epad, q, last)

    pool = params["pool"]
    fu = params["fusion"]
    W1 = fu["W1"]
    out = _fusion_call(
        state[0], state[1],
        pool["W1"].T, pool["b1"].reshape(1, -1),
        pool["W2"].T, pool["b2"].reshape(1, -1),
        W1[:, :OC].T, W1[:, OC:2 * OC].T, W1[:, 2 * OC:].T,
        fu["b1"].reshape(1, -1), fu["W2"].T, fu["b2"].reshape(1, -1))
    return out[:N]


# speedup vs baseline: 54.9838x; 1.0003x over previous
import functools

import jax
import jax.numpy as jnp
from jax import lax
from jax.experimental import pallas as pl
from jax.experimental.pallas import tpu as pltpu
from jax.experimental.pallas import tpu_sc as plsc

N = 10000
NPAD = 10240
HEADS = 8
OC = 64
HD = HEADS * OC
NG = 4
C = 128
C1 = 256
NTILES = 32
ROWS_PER_TILE = NPAD // 16
F32 = jnp.float32


def _cfgs_for(num_layers, in_ch=128, hid=64, out=64, heads=HEADS):
    cfgs = []
    cur = in_ch
    for i in range(num_layers):
        last = (i == num_layers - 1)
        oc = out if last else hid
        concat = (not last)
        da = oc * heads if concat else oc
        cfgs.append(dict(in_dim=cur, out_ch=oc, heads=heads, concat=concat,
                         dim_after=da, has_proj=(cur != da)))
        cur = da
    return cfgs


def _pre_call(xp, W, att_s, att_d, projWt, projb):
    ind = xp.shape[1]
    has_proj = projWt is not None
    BN = 1280
    grid = (NPAD // BN,)

    def body(x_ref, w_ref, as_ref, ad_ref, *rest):
        if has_proj:
            pw_ref, pb_ref = rest[0], rest[1]
            outs = rest[2:]
        else:
            outs = rest
        h0, h1, h2, h3, abs_ref, abd_ref = outs[:6]
        xb = x_ref[...]
        hb = jnp.dot(xb, w_ref[...], preferred_element_type=F32)
        acols, bcols = [], []
        for k in range(HEADS):
            hk = hb[:, OC * k:OC * (k + 1)]
            acols.append(jnp.sum(hk * as_ref[:, OC * k:OC * (k + 1)], axis=1,
                                 keepdims=True))
            bcols.append(jnp.sum(hk * ad_ref[:, OC * k:OC * (k + 1)], axis=1,
                                 keepdims=True))
        zeros8 = jnp.zeros((BN, 8), F32)
        abs_ref[...] = jnp.concatenate(acols + [zeros8], axis=1)
        abd_ref[...] = jnp.concatenate(bcols + [zeros8], axis=1)
        for g, hg in enumerate((h0, h1, h2, h3)):
            hg[...] = hb[:, 128 * g:128 * (g + 1)]
        if has_proj:
            da = pw_ref.shape[1]
            outs[6][...] = jnp.dot(xb, pw_ref[...],
                                   preferred_element_type=F32) + pb_ref[...]

    in_specs = [
        pl.BlockSpec((BN, ind), lambda i: (i, 0)),
        pl.BlockSpec((ind, HD), lambda i: (0, 0)),
        pl.BlockSpec((1, HD), lambda i: (0, 0)),
        pl.BlockSpec((1, HD), lambda i: (0, 0)),
    ]
    args = [xp, W, att_s, att_d]
    out_shape = [jax.ShapeDtypeStruct((NPAD, 128), F32) for _ in range(4)]
    out_shape += [jax.ShapeDtypeStruct((NPAD, 16), F32) for _ in range(2)]
    out_specs = [pl.BlockSpec((BN, 128), lambda i: (i, 0)) for _ in range(4)]
    out_specs += [pl.BlockSpec((BN, 16), lambda i: (i, 0)) for _ in range(2)]
    if has_proj:
        da = projWt.shape[1]
        in_specs += [pl.BlockSpec((ind, da), lambda i: (0, 0)),
                     pl.BlockSpec((1, da), lambda i: (0, 0))]
        args += [projWt, projb]
        out_shape.append(jax.ShapeDtypeStruct((NPAD, da), F32))
        out_specs.append(pl.BlockSpec((BN, da), lambda i: (i, 0)))
    return pl.pallas_call(
        body, grid=grid, in_specs=in_specs, out_specs=out_specs,
        out_shape=out_shape)(*args)


def _dyn_gather(row, idx):
    return lax.gather(
        row, idx[:, None],
        lax.GatherDimensionNumbers(offset_dims=(), collapsed_slice_dims=(0,),
                                   start_index_map=(0,)),
        slice_sizes=(1,), mode=lax.GatherScatterMode.PROMISE_IN_BOUNDS)


def _weights_call(ab_s, ab_d, srcp, dstp, epad, q1):
    mesh = plsc.VectorSubcoreMesh(core_axis_name="c", subcore_axis_name="s")
    out_type = [
        jax.ShapeDtypeStruct((2, NPAD, 16), F32),
        jax.ShapeDtypeStruct((epad, 16), F32),
    ]
    scratch = [
        pltpu.VMEM((C1,), jnp.int32),
        pltpu.VMEM((C1,), jnp.int32),
        pltpu.VMEM((C1,), jnp.int32),
        pltpu.VMEM((C1,), jnp.int32),
        pltpu.VMEM((C1, 16), F32),
        pltpu.VMEM((C1, 16), F32),
        pltpu.VMEM((C1, 16), F32),
        pltpu.VMEM((C1, 16), F32),
        pltpu.SemaphoreType.DMA,
        pltpu.SemaphoreType.DMA,
        pltpu.SemaphoreType.DMA,
        pltpu.SemaphoreType.DMA,
        pltpu.SemaphoreType.DMA,
        pltpu.VMEM_SHARED((NPAD, 16), F32),
    ]

    @functools.partial(pl.kernel, mesh=mesh, out_type=out_type,
                       scratch_types=scratch,
                       compiler_params=pltpu.CompilerParams(
                           use_tc_tiling_on_sc=False))
    def k(absr, abdr, srcr, dstr, ssum, wg,
          is0, id0, is1, id1, S0, S1, D0, D1,
          isem, gsem, ssem, dsem, wsem, s_sp):
        cid = lax.axis_index("c")
        sid = lax.axis_index("s")
        t = cid * 16 + sid
        r0 = sid * ROWS_PER_TILE

        @plsc.parallel_loop(0, C1, unroll=4)
        def _(e):
            S0[e, :] = jnp.zeros((16,), F32)
        off = 0
        while off < ROWS_PER_TILE:
            n = min(C1, ROWS_PER_TILE - off)
            pltpu.sync_copy(S0.at[pl.ds(0, n)], s_sp.at[pl.ds(r0 + off, n)])
            off += n
        plsc.subcore_barrier()

        bufs = [(is0, id0, S0, D0), (is1, id1, S1, D1)]

        base0 = (t * q1) * C1
        pltpu.sync_copy(srcr.at[pl.ds(base0, C1)], is0)
        pltpu.sync_copy(dstr.at[pl.ds(base0, C1)], id0)
        pltpu.async_copy(absr.at[is0], S0, gsem)
        pltpu.async_copy(abdr.at[id0], D0, gsem)
        pltpu.async_copy(srcr.at[pl.ds(base0 + C1, C1)], is1, isem)

        def step(i, cur, nxt):
            @pl.when(i >= 1)
            def _():
                basep = (t * q1 + i - 1) * C1
                pltpu.make_async_copy(nxt[2], s_sp.at[nxt[1]], ssem).wait()
                pltpu.make_async_copy(nxt[2], wg.at[pl.ds(basep, C1)],
                                      wsem).wait()

            @pl.when(i + 1 < q1)
            def _():
                base = (t * q1 + i + 1) * C1
                pltpu.make_async_copy(srcr.at[pl.ds(base, C1)], nxt[0],
                                      isem).wait()
                pltpu.async_copy(dstr.at[pl.ds(base, C1)], nxt[1], dsem)
                pltpu.async_copy(absr.at[nxt[0]], nxt[2], gsem)

            pltpu.make_async_copy(absr.at[cur[0]], cur[2], gsem).wait()
            pltpu.make_async_copy(abdr.at[cur[1]], cur[3], gsem).wait()

            S, D = cur[2], cur[3]

            @plsc.parallel_loop(0, C1, unroll=8)
            def _(e):
                z = S[e, :] + D[e, :]
                S[e, :] = jnp.exp(jnp.maximum(z, 0.2 * z))

            @pl.when(i + 1 < q1)
            def _():
                base = (t * q1 + i + 1) * C1
                pltpu.make_async_copy(dstr.at[pl.ds(base, C1)], nxt[1],
                                      dsem).wait()
                pltpu.async_copy(abdr.at[nxt[1]], nxt[3], gsem)

            base_i = (t * q1 + i) * C1
            pltpu.async_copy(S, s_sp.at[cur[1]], ssem, add=True)
            pltpu.async_copy(S, wg.at[pl.ds(base_i, C1)], wsem)

            @pl.when(i + 2 < q1)
            def _():
                base = (t * q1 + i + 2) * C1
                pltpu.async_copy(srcr.at[pl.ds(base, C1)], cur[0], isem)

        def loop(ip, _):
            step(2 * ip, bufs[0], bufs[1])
            step(2 * ip + 1, bufs[1], bufs[0])
            return 0
        lax.fori_loop(0, q1 // 2, loop, 0)
        last = bufs[(q1 - 1) % 2]
        basel = (t * q1 + q1 - 1) * C1
        pltpu.make_async_copy(last[2], s_sp.at[last[1]], ssem).wait()
        pltpu.make_async_copy(last[2], wg.at[pl.ds(basel, C1)], wsem).wait()
        plsc.subcore_barrier()
        pltpu.sync_copy(s_sp.at[pl.ds(r0, ROWS_PER_TILE)],
                        ssum.at[cid, pl.ds(r0, ROWS_PER_TILE)])

    return k(ab_s, ab_d, srcp, dstp)


def _agg_call(hgs, wg, srcp, dstp, epad, q):
    mesh = plsc.VectorSubcoreMesh(core_axis_name="c", subcore_axis_name="s")
    out_type = [
        jax.ShapeDtypeStruct((2, NG, NPAD, 128), F32),
    ]
    scratch = [
        pltpu.VMEM((C,), jnp.int32),
        pltpu.VMEM((C,), jnp.int32),
        pltpu.VMEM((C,), jnp.int32),
        pltpu.VMEM((C,), jnp.int32),
        pltpu.VMEM((C, 16), F32),
        pltpu.VMEM((C, 16), F32),
        pltpu.VMEM((C, 128), F32),
        pltpu.VMEM((C, 128), F32),
        pltpu.SemaphoreType.DMA,
        pltpu.SemaphoreType.DMA,
        pltpu.SemaphoreType.DMA,
        pltpu.SemaphoreType.DMA,
        pltpu.VMEM_SHARED((NPAD, 128), F32),
    ]

    @functools.partial(pl.kernel, mesh=mesh, out_type=out_type,
                       scratch_types=scratch,
                       compiler_params=pltpu.CompilerParams(
                           use_tc_tiling_on_sc=False))
    def k(h0, h1, h2, h3, wg, srcr, dstr,
          accs,
          idxs0, idxd0, idxs1, idxd1, Wv0, Wv1, rows0, rows1,
          isem, gsem, ssem, dsem, acc_sp):
        cid = lax.axis_index("c")
        sid = lax.axis_index("s")
        t = cid * 16 + sid
        r0 = sid * ROWS_PER_TILE

        bufs = [(idxs0, idxd0, Wv0, rows0), (idxs1, idxd1, Wv1, rows1)]

        def _prefetch(i, b):
            base = (t * q + i) * C
            pltpu.async_copy(srcr.at[pl.ds(base, C)], b[0], isem)
            pltpu.async_copy(wg.at[pl.ds(base, C)], b[2], isem)

        def _prefetch_wait(i, b):
            base = (t * q + i) * C
            pltpu.make_async_copy(srcr.at[pl.ds(base, C)], b[0], isem).wait()
            pltpu.make_async_copy(wg.at[pl.ds(base, C)], b[2], isem).wait()

        for g in range(NG):
            hg = (h0, h1, h2, h3)[g]

            def zr(e, _):
                for j in range(8):
                    rows0[e, pl.ds(16 * j, 16)] = jnp.zeros((16,), F32)
                return 0
            lax.fori_loop(0, C, zr, 0)
            for part in range(ROWS_PER_TILE // C):
                pltpu.sync_copy(rows0, acc_sp.at[pl.ds(r0 + C * part, C)])
            plsc.subcore_barrier()

            gi0 = jnp.full((16,), 2 * g, jnp.int32)
            gi1 = jnp.full((16,), 2 * g + 1, jnp.int32)

            base0 = (t * q) * C
            pltpu.sync_copy(srcr.at[pl.ds(base0, C)], idxs0)
            pltpu.sync_copy(wg.at[pl.ds(base0, C)], Wv0)
            pltpu.async_copy(dstr.at[pl.ds(base0, C)], idxd0, dsem)
            pltpu.async_copy(hg.at[idxs0], rows0, gsem)
            _prefetch(1, bufs[1])

            def step(i, cur, nxt):
                @pl.when(i + 1 < q)
                def _():
                    _prefetch_wait(i + 1, nxt)

                @pl.when(i >= 1)
                def _():
                    pltpu.make_async_copy(nxt[3], acc_sp.at[nxt[1]],
                                          ssem).wait()

                @pl.when(i + 1 < q)
                def _():
                    base = (t * q + i + 1) * C
                    pltpu.async_copy(dstr.at[pl.ds(base, C)], nxt[1], dsem)
                pltpu.make_async_copy(hg.at[cur[0]], cur[3], gsem).wait()

                @pl.when(i + 1 < q)
                def _():
                    pltpu.async_copy(hg.at[nxt[0]], nxt[3], gsem)

                wv, rows = cur[2], cur[3]

                @plsc.parallel_loop(0, C, unroll=8)
                def _(e):
                    wrow = wv[e, :]
                    b0 = _dyn_gather(wrow, gi0)
                    b1 = _dyn_gather(wrow, gi1)
                    for j in range(4):
                        rows[e, pl.ds(16 * j, 16)] = \
                            rows[e, pl.ds(16 * j, 16)] * b0
                    for j in range(4, 8):
                        rows[e, pl.ds(16 * j, 16)] = \
                            rows[e, pl.ds(16 * j, 16)] * b1
                base_i = (t * q + i) * C
                pltpu.make_async_copy(dstr.at[pl.ds(base_i, C)], cur[1],
                                      dsem).wait()
                pltpu.async_copy(rows, acc_sp.at[cur[1]], ssem, add=True)

                @pl.when(i + 2 < q)
                def _():
                    _prefetch(i + 2, cur)

            def chunk2(ip, _):
                step(2 * ip, bufs[0], bufs[1])
                step(2 * ip + 1, bufs[1], bufs[0])
                return 0
            lax.fori_loop(0, q // 2, chunk2, 0)
            pltpu.make_async_copy(bufs[(q - 1) % 2][3],
                                  acc_sp.at[bufs[(q - 1) % 2][1]],
                                  ssem).wait()
            plsc.subcore_barrier()
            pltpu.sync_copy(acc_sp.at[pl.ds(r0, ROWS_PER_TILE)],
                            accs.at[cid, g, pl.ds(r0, ROWS_PER_TILE)])

    return k(hgs[0], hgs[1], hgs[2], hgs[3], wg, srcp, dstp)[0]


def _edge_call(hgs, ab_s, ab_d, srcp, dstp, epad, q):
    ssum, wg = _weights_call(ab_s, ab_d, srcp, dstp, epad, q * C // C1)
    accs = _agg_call(hgs, wg, srcp, dstp, epad, q)
    return accs, ssum


def _post_call(accs, ssum, proj, bias, ln_g, ln_b, sw, concat, last):
    da = HD if concat else OC
    BN = 1280
    grid = (NPAD // BN,)

    def body(accs_ref, ssum_ref, proj_ref, bias_ref, g_ref, b_ref, sw_ref,
             out_ref):
        s = ssum_ref[0] + ssum_ref[1]
        cols = []
        for k in range(HEADS):
            g, m = k // 2, k % 2
            a = (accs_ref[0, g][:, OC * m:OC * (m + 1)]
                 + accs_ref[1, g][:, OC * m:OC * (m + 1)])
            den = s[:, k:k + 1] + 1e-16
            cols.append(a / den)
        if concat:
            xg = jnp.concatenate(cols, axis=1)
        else:
            acc = cols[0]
            for ck in cols[1:]:
                acc = acc + ck
            xg = acc / float(HEADS)
        xg = xg + bias_ref[...]
        mu = jnp.mean(xg, axis=1, keepdims=True)
        var = jnp.mean((xg - mu) * (xg - mu), axis=1, keepdims=True)
        xn = (xg - mu) * lax.rsqrt(var + 1e-5) * g_ref[...] + b_ref[...]
        res = xn + sw_ref[...] * proj_ref[...]
        if last:
            out_ref[...] = res
        else:
            out_ref[...] = jnp.where(res > 0, res, jnp.exp(res) - 1.0)

    in_specs = [
        pl.BlockSpec((2, NG, BN, 128), lambda i: (0, 0, i, 0)),
        pl.BlockSpec((2, BN, 16), lambda i: (0, i, 0)),
        pl.BlockSpec((BN, da), lambda i: (i, 0)),
        pl.BlockSpec((1, da), lambda i: (0, 0)),
        pl.BlockSpec((1, da), lambda i: (0, 0)),
        pl.BlockSpec((1, da), lambda i: (0, 0)),
        pl.BlockSpec((1, 1), lambda i: (0, 0)),
    ]
    return pl.pallas_call(
        body, grid=grid, in_specs=in_specs,
        out_specs=pl.BlockSpec((BN, da), lambda i: (i, 0)),
        out_shape=jax.ShapeDtypeStruct((NPAD, da), F32),
    )(accs, ssum, proj, bias, ln_g, ln_b, sw)


def _fusion_call(f0, f1, pw1t, pb1, pw2t, pb2, wa, wb, wgc, fb1, w2t, fb2):
    def body(f0r, f1r, pw1r, pb1r, pw2r, pb2r, war, wbr, wgr, fb1r, w2r,
             fb2r, out_ref):
        x0 = f0r[...]
        t1 = jnp.tanh(jnp.dot(x0, pw1r[...], preferred_element_type=F32)
                      + pb1r[...])
        sc = jnp.dot(t1, pw2r[...], preferred_element_type=F32) + pb2r[...]
        rid = lax.broadcasted_iota(jnp.int32, (NPAD, 1), 0)
        valid = rid < N
        scm = jnp.where(valid, sc, jnp.full_like(sc, -1e30))
        m = jnp.max(scm)
        ex = jnp.where(valid, jnp.exp(sc - m), jnp.zeros_like(sc))
        wgt = ex / jnp.sum(ex)
        gvec = jnp.sum(x0 * wgt, axis=0, keepdims=True)
        h1 = (jnp.dot(x0, war[...], preferred_element_type=F32)
              + jnp.dot(f1r[...], wbr[...], preferred_element_type=F32)
              + jnp.dot(gvec, wgr[...], preferred_element_type=F32)
              + fb1r[...])
        h1 = jnp.maximum(h1, 0.0)
        out_ref[...] = jnp.dot(h1, w2r[...],
                               preferred_element_type=F32) + fb2r[...]

    return pl.pallas_call(
        body,
        out_shape=jax.ShapeDtypeStruct((NPAD, 128), F32),
    )(f0, f1, pw1t, pb1, pw2t, pb2, wa, wb, wgc, fb1, w2t, fb2)


def _layer(h_in, p, c, srcp, dstp, epad, q, last):
    att_s = p["att_src"].reshape(1, HD)
    att_d = p["att_dst"].reshape(1, HD)
    if c["has_proj"]:
        projWt = p["proj_W"].T
        projb = p["proj_b"].reshape(1, -1)
        pre = _pre_call(h_in, p["W"], att_s, att_d, projWt, projb)
        hgs, ab_s, ab_d, proj = pre[:4], pre[4], pre[5], pre[6]
    else:
        pre = _pre_call(h_in, p["W"], att_s, att_d, None, None)
        hgs, ab_s, ab_d, proj = pre[:4], pre[4], pre[5], h_in
    accs, ssum = _edge_call(hgs, ab_s, ab_d, srcp, dstp, epad, q)
    sw = jax.nn.sigmoid(p["res_w"]).reshape(1, 1)
    return _post_call(accs, ssum, proj, p["bias"].reshape(1, -1),
                      p["ln_g"].reshape(1, -1), p["ln_b"].reshape(1, -1),
                      sw, c["concat"], last)


def kernel(x, params, edge_index):
    xp = jnp.pad(x, ((0, NPAD - N), (0, 0)))
    loop = jnp.arange(N, dtype=edge_index.dtype)
    src = jnp.concatenate([edge_index[0], loop]).astype(jnp.int32)
    dst = jnp.concatenate([edge_index[1], loop]).astype(jnp.int32)
    ne = src.shape[0]
    q = -(-ne // (NTILES * C))
    q = q + (-q) % 4
    epad = NTILES * q * C
    npe = epad - ne
    pad_idx = (N + (jnp.arange(npe, dtype=jnp.int32) % 16)).astype(jnp.int32)
    srcp = jnp.concatenate([src, pad_idx])
    dstp = jnp.concatenate([dst, pad_idx])

    encs = params["encoders"]
    cfgs = [_cfgs_for(2), _cfgs_for(3)]
    state = [xp, xp]
    for li in range(3):
        for enc in range(2):
            if li >= len(cfgs[enc]):
                continue
            c = cfgs[enc][li]
            last = (li == len(cfgs[enc]) - 1)
            state[enc] = _layer(state[enc], encs[enc][li], c, srcp, dstp,
                                epad, q, last)

    pool = params["pool"]
    fu = params["fusion"]
    W1 = fu["W1"]
    out = _fusion_call(
        state[0], state[1],
        pool["W1"].T, pool["b1"].reshape(1, -1),
        pool["W2"].T, pool["b2"].reshape(1, -1),
        W1[:, :OC].T, W1[:, OC:2 * OC].T, W1[:, 2 * OC:].T,
        fu["b1"].reshape(1, -1), fu["W2"].T, fu["b2"].reshape(1, -1))
    return out[:N]
